# Initial kernel scaffold; baseline (speedup 1.0000x reference)
import functools

import jax
import jax.numpy as jnp
from jax import lax
from jax.experimental import pallas as pl
from jax.experimental.pallas import tpu as pltpu
from jax.experimental.pallas import tpu_sc as plsc

N = 100000
E = 1600000
D_IN = 3
DH = 16

NACC = 102400
C = 128
NW = 32
EPAD = 1638400
NCH = EPAD // (NW * C)
EPT = EPAD // NW
RPT = NACC // 16

_mesh = plsc.VectorSubcoreMesh(core_axis_name="c", subcore_axis_name="s")


@functools.partial(
    pl.kernel,
    mesh=_mesh,
    out_type=(
        jax.ShapeDtypeStruct((2, NACC, DH), jnp.float32),
        jax.ShapeDtypeStruct((NW, NACC), jnp.float32),
    ),
    scratch_types=[
        pltpu.VMEM((C,), jnp.int32),
        pltpu.VMEM((C,), jnp.int32),
        pltpu.VMEM((C,), jnp.float32),
        pltpu.VMEM((C, DH), jnp.float32),
        pltpu.VMEM((NACC,), jnp.float32),
        pltpu.VMEM_SHARED((NACC, DH), jnp.float32),
        pltpu.SemaphoreType.DMA,
    ],
)
def _edge_pass_deg(h_hbm, src_hbm, dst_hbm, w_hbm, z2_hbm,
                   acc_out, deg_out,
                   src_v, dst_v, w_v, rows_v, deg_v, acc_sh, sem):
    c = lax.axis_index("c")
    s = lax.axis_index("s")
    wid = c * 16 + s
    pltpu.sync_copy(z2_hbm.at[pl.ds(s * RPT, RPT)],
                    acc_sh.at[pl.ds(s * RPT, RPT)])

    def _zb(j, carry):
        deg_v[pl.ds(j * 16, 16)] = jnp.zeros((16,), jnp.float32)
        return carry
    lax.fori_loop(0, NACC // 16, _zb, 0)
    plsc.subcore_barrier()

    ebase = wid * EPT

    def _chunk(ch, carry):
        base = ebase + ch * C
        pltpu.sync_copy(src_hbm.at[pl.ds(base, C)], src_v)
        pltpu.sync_copy(dst_hbm.at[pl.ds(base, C)], dst_v)
        pltpu.sync_copy(w_hbm.at[pl.ds(base, C)], w_v)
        pltpu.async_copy(h_hbm.at[src_v], rows_v, sem).wait()

        def _mul(e, cc):
            rows_v[e, :] = rows_v[e, :] * w_v[e]
            return cc
        lax.fori_loop(0, C, _mul, 0)
        pltpu.sync_copy(rows_v, acc_sh.at[dst_v], add=True)

        def _dh(k, cc):
            idx = dst_v[pl.ds(k * 16, 16)]
            plsc.addupdate_scatter(deg_v, [idx],
                                   jnp.ones((16,), jnp.float32))
            return cc
        lax.fori_loop(0, C // 16, _dh, 0)
        return carry
    lax.fori_loop(0, NCH, _chunk, 0)
    plsc.subcore_barrier()
    pltpu.sync_copy(acc_sh.at[pl.ds(s * RPT, RPT)],
                    acc_out.at[c, pl.ds(s * RPT, RPT)])
    pltpu.sync_copy(deg_v, deg_out.at[wid])


@functools.partial(
    pl.kernel,
    mesh=_mesh,
    out_type=jax.ShapeDtypeStruct((2, NACC, DH), jnp.float32),
    scratch_types=[
        pltpu.VMEM((C,), jnp.int32),
        pltpu.VMEM((C,), jnp.int32),
        pltpu.VMEM((C,), jnp.float32),
        pltpu.VMEM((C, DH), jnp.float32),
        pltpu.VMEM_SHARED((NACC, DH), jnp.float32),
        pltpu.SemaphoreType.DMA,
    ],
)
def _edge_pass(h_hbm, src_hbm, dst_hbm, w_hbm, z2_hbm,
               acc_out,
               src_v, dst_v, w_v, rows_v, acc_sh, sem):
    c = lax.axis_index("c")
    s = lax.axis_index("s")
    pltpu.sync_copy(z2_hbm.at[pl.ds(s * RPT, RPT)],
                    acc_sh.at[pl.ds(s * RPT, RPT)])
    plsc.subcore_barrier()

    ebase = (c * 16 + s) * EPT

    def _chunk(ch, carry):
        base = ebase + ch * C
        pltpu.sync_copy(src_hbm.at[pl.ds(base, C)], src_v)
        pltpu.sync_copy(dst_hbm.at[pl.ds(base, C)], dst_v)
        pltpu.sync_copy(w_hbm.at[pl.ds(base, C)], w_v)
        pltpu.async_copy(h_hbm.at[src_v], rows_v, sem).wait()

        def _mul(e, cc):
            rows_v[e, :] = rows_v[e, :] * w_v[e]
            return cc
        lax.fori_loop(0, C, _mul, 0)
        pltpu.sync_copy(rows_v, acc_sh.at[dst_v], add=True)
        return carry
    lax.fori_loop(0, NCH, _chunk, 0)
    plsc.subcore_barrier()
    pltpu.sync_copy(acc_sh.at[pl.ds(s * RPT, RPT)],
                    acc_out.at[c, pl.ds(s * RPT, RPT)])


BR = 800


def _combine_body(sig, p_ref, degp_ref, h_ref, ws_ref, wn_ref, b_ref, o_ref):
    deg = jnp.sum(degp_ref[...], axis=0)
    recip = 1.0 / jnp.maximum(deg, 1.0)
    neigh = (p_ref[0] + p_ref[1]) * recip[:, None]
    z = (jnp.dot(h_ref[...], ws_ref[...], preferred_element_type=jnp.float32)
         + jnp.dot(neigh, wn_ref[...], preferred_element_type=jnp.float32)
         + b_ref[...])
    if sig:
        z = 1.0 / (1.0 + jnp.exp(-z))
    o_ref[...] = z


def _combine(p, degp, h, ws, wn, b, sig):
    return pl.pallas_call(
        functools.partial(_combine_body, sig),
        grid=(NACC // BR,),
        in_specs=[
            pl.BlockSpec((2, BR, DH), lambda i: (0, i, 0)),
            pl.BlockSpec((NW, BR), lambda i: (0, i)),
            pl.BlockSpec((BR, DH), lambda i: (i, 0)),
            pl.BlockSpec((DH, DH), lambda i: (0, 0)),
            pl.BlockSpec((DH, DH), lambda i: (0, 0)),
            pl.BlockSpec((1, DH), lambda i: (0, 0)),
        ],
        out_specs=pl.BlockSpec((BR, DH), lambda i: (i, 0)),
        out_shape=jax.ShapeDtypeStruct((NACC, DH), jnp.float32),
    )(p, degp, h, ws, wn, b)


def _readout_body(p_ref, degp_ref, h_ref, ws_ref, wn_ref, b_ref,
                  f_ref, wrf_ref, wrh_ref, bro_ref, o_ref):
    deg = jnp.sum(degp_ref[...], axis=0)
    recip = 1.0 / jnp.maximum(deg, 1.0)
    neigh = (p_ref[0] + p_ref[1]) * recip[:, None]
    z = (jnp.dot(h_ref[...], ws_ref[...], preferred_element_type=jnp.float32)
         + jnp.dot(neigh, wn_ref[...], preferred_element_type=jnp.float32)
         + b_ref[...])
    o_ref[...] = (jnp.dot(f_ref[...], wrf_ref[...],
                          preferred_element_type=jnp.float32)
                  + jnp.dot(z, wrh_ref[...],
                            preferred_element_type=jnp.float32)
                  + bro_ref[...])


def _readout(p, degp, h, ws, wn, b, feats, wrf, wrh, bro):
    return pl.pallas_call(
        _readout_body,
        grid=(N // BR,),
        in_specs=[
            pl.BlockSpec((2, BR, DH), lambda i: (0, i, 0)),
            pl.BlockSpec((NW, BR), lambda i: (0, i)),
            pl.BlockSpec((BR, DH), lambda i: (i, 0)),
            pl.BlockSpec((DH, DH), lambda i: (0, 0)),
            pl.BlockSpec((DH, DH), lambda i: (0, 0)),
            pl.BlockSpec((1, DH), lambda i: (0, 0)),
            pl.BlockSpec((BR, D_IN), lambda i: (i, 0)),
            pl.BlockSpec((D_IN, 1), lambda i: (0, 0)),
            pl.BlockSpec((DH, 1), lambda i: (0, 0)),
            pl.BlockSpec((1, 1), lambda i: (0, 0)),
        ],
        out_specs=pl.BlockSpec((BR, 1), lambda i: (i, 0)),
        out_shape=jax.ShapeDtypeStruct((N, 1), jnp.float32),
    )(p, degp, h, ws, wn, b, feats, wrf, wrh, bro)


def kernel(features, edge_index, e_feat,
           W_self0, W_neigh0, b0,
           W_self1, W_neigh1, b1,
           W_self2, W_neigh2, b2,
           W_ro, b_ro):
    src = edge_index[0]
    dst = edge_index[1]
    pad = EPAD - E
    ar = jnp.arange(pad, dtype=jnp.int32)
    src_p = jnp.concatenate([src, ar % N])
    dst_p = jnp.concatenate([dst, N + ar % (NACC - N)])
    w_p = jnp.concatenate([e_feat[:, 0], jnp.zeros((pad,), jnp.float32)])

    h0 = jnp.concatenate([
        jnp.pad(features, ((0, NACC - N), (0, 0))),
        jnp.ones((NACC, 7), jnp.float32),
        jnp.zeros((NACC, DH - D_IN - 7), jnp.float32)], axis=1)
    z2d = jnp.zeros((NACC, DH), jnp.float32)
    Ws0 = jnp.pad(W_self0, ((0, DH - D_IN - 7), (0, 0)))
    Wn0 = jnp.pad(W_neigh0, ((0, DH - D_IN - 7), (0, 0)))

    acc0, degp = _edge_pass_deg(h0, src_p, dst_p, w_p, z2d)
    h1 = _combine(acc0, degp, h0, Ws0, Wn0, b0.reshape(1, DH), sig=True)
    acc1 = _edge_pass(h1, src_p, dst_p, w_p, z2d)
    h2 = _combine(acc1, degp, h1, W_self1, W_neigh1, b1.reshape(1, DH),
                  sig=True)
    acc2 = _edge_pass(h2, src_p, dst_p, w_p, z2d)
    out = _readout(acc2, degp, h2, W_self2, W_neigh2, b2.reshape(1, DH),
                   features, W_ro[:D_IN], W_ro[D_IN:], b_ro.reshape(1, 1))
    return out


# speedup vs baseline: 6.3384x; 6.3384x over previous
import functools

import jax
import jax.numpy as jnp
from jax import lax
from jax.experimental import pallas as pl
from jax.experimental.pallas import tpu as pltpu
from jax.experimental.pallas import tpu_sc as plsc

N = 100000
E = 1600000
D_IN = 3
DH = 16

NACC = 102400
C = 128
NW = 32
EPAD = 1638400
NCH = EPAD // (NW * C)
EPT = EPAD // NW
RPT = NACC // 16

@functools.cache
def _build_edge_pass_deg():
  return functools.partial(
      pl.kernel,
      mesh=plsc.VectorSubcoreMesh(core_axis_name="c", subcore_axis_name="s"),
      compiler_params=pltpu.CompilerParams(use_tc_tiling_on_sc=False),
      out_type=(
          jax.ShapeDtypeStruct((2, NACC, DH), jnp.float32),
          jax.ShapeDtypeStruct((2, NACC), jnp.float32),
      ),
      scratch_types=[
          pltpu.VMEM((C,), jnp.int32),
          pltpu.VMEM((C,), jnp.int32),
          pltpu.VMEM((C,), jnp.float32),
          pltpu.VMEM((C, DH), jnp.float32),
          pltpu.VMEM((C,), jnp.float32),
          pltpu.VMEM_SHARED((NACC, DH), jnp.float32),
          pltpu.VMEM_SHARED((NACC,), jnp.float32),
          pltpu.SemaphoreType.DMA,
      ],
  )(_edge_pass_deg_body)


def _edge_pass_deg(*args):
  return _build_edge_pass_deg()(*args)


def _edge_pass_deg_body(h_hbm, src_hbm, dst_hbm, w_hbm, z2_hbm, z1_hbm,
                        acc_out, deg_out,
                        src_v, dst_v, w_v, rows_v, ones_v, acc_sh, deg_sh,
                        sem):
    c = lax.axis_index("c")
    s = lax.axis_index("s")
    wid = c * 16 + s
    pltpu.sync_copy(z2_hbm.at[pl.ds(s * RPT, RPT)],
                    acc_sh.at[pl.ds(s * RPT, RPT)])
    pltpu.sync_copy(z1_hbm.at[pl.ds(s * RPT, RPT)],
                    deg_sh.at[pl.ds(s * RPT, RPT)])

    def _ob(j, carry):
        ones_v[pl.ds(j * 16, 16)] = jnp.ones((16,), jnp.float32)
        return carry
    lax.fori_loop(0, C // 16, _ob, 0)
    plsc.subcore_barrier()

    ebase = wid * EPT

    def _chunk(ch, carry):
        base = ebase + ch * C
        pltpu.sync_copy(src_hbm.at[pl.ds(base, C)], src_v)
        pltpu.sync_copy(dst_hbm.at[pl.ds(base, C)], dst_v)
        pltpu.sync_copy(w_hbm.at[pl.ds(base, C)], w_v)
        pltpu.async_copy(h_hbm.at[src_v], rows_v, sem).wait()

        def _mul(g, cc):
            wv = w_v[pl.ds(g * 16, 16)]
            for e in range(16):
                r = g * 16 + e
                rows_v[r, :] = rows_v[r, :] * wv[e]
            return cc
        lax.fori_loop(0, C // 16, _mul, 0)
        pltpu.sync_copy(rows_v, acc_sh.at[dst_v], add=True)

        pltpu.sync_copy(ones_v, deg_sh.at[dst_v], add=True)
        return carry
    lax.fori_loop(0, NCH, _chunk, 0)
    plsc.subcore_barrier()
    pltpu.sync_copy(acc_sh.at[pl.ds(s * RPT, RPT)],
                    acc_out.at[c, pl.ds(s * RPT, RPT)])
    pltpu.sync_copy(deg_sh.at[pl.ds(s * RPT, RPT)],
                    deg_out.at[c, pl.ds(s * RPT, RPT)])


@functools.cache
def _build_edge_pass():
  return functools.partial(
      pl.kernel,
      mesh=plsc.VectorSubcoreMesh(core_axis_name="c", subcore_axis_name="s"),
      compiler_params=pltpu.CompilerParams(use_tc_tiling_on_sc=False),
      out_type=jax.ShapeDtypeStruct((2, NACC, DH), jnp.float32),
      scratch_types=[
          pltpu.VMEM((C,), jnp.int32),
          pltpu.VMEM((C,), jnp.int32),
          pltpu.VMEM((C,), jnp.float32),
          pltpu.VMEM((C, DH), jnp.float32),
          pltpu.VMEM_SHARED((NACC, DH), jnp.float32),
          pltpu.SemaphoreType.DMA,
      ],
  )(_edge_pass_body)


def _edge_pass(*args):
  return _build_edge_pass()(*args)


def _edge_pass_body(h_hbm, src_hbm, dst_hbm, w_hbm, z2_hbm,
                    acc_out,
                    src_v, dst_v, w_v, rows_v, acc_sh, sem):
    c = lax.axis_index("c")
    s = lax.axis_index("s")
    pltpu.sync_copy(z2_hbm.at[pl.ds(s * RPT, RPT)],
                    acc_sh.at[pl.ds(s * RPT, RPT)])
    plsc.subcore_barrier()

    ebase = (c * 16 + s) * EPT

    def _chunk(ch, carry):
        base = ebase + ch * C
        pltpu.sync_copy(src_hbm.at[pl.ds(base, C)], src_v)
        pltpu.sync_copy(dst_hbm.at[pl.ds(base, C)], dst_v)
        pltpu.sync_copy(w_hbm.at[pl.ds(base, C)], w_v)
        pltpu.async_copy(h_hbm.at[src_v], rows_v, sem).wait()

        def _mul(g, cc):
            wv = w_v[pl.ds(g * 16, 16)]
            for e in range(16):
                r = g * 16 + e
                rows_v[r, :] = rows_v[r, :] * wv[e]
            return cc
        lax.fori_loop(0, C // 16, _mul, 0)
        pltpu.sync_copy(rows_v, acc_sh.at[dst_v], add=True)
        return carry
    lax.fori_loop(0, NCH, _chunk, 0)
    plsc.subcore_barrier()
    pltpu.sync_copy(acc_sh.at[pl.ds(s * RPT, RPT)],
                    acc_out.at[c, pl.ds(s * RPT, RPT)])


BR = 1024


def _combine_body(sig, p_ref, degp_ref, h_ref, ws_ref, wn_ref, b_ref, o_ref):
    deg = jnp.sum(degp_ref[...], axis=0)
    recip = 1.0 / jnp.maximum(deg, 1.0)
    neigh = (p_ref[0] + p_ref[1]) * recip[:, None]
    z = (jnp.dot(h_ref[...], ws_ref[...], preferred_element_type=jnp.float32)
         + jnp.dot(neigh, wn_ref[...], preferred_element_type=jnp.float32)
         + b_ref[...])
    if sig:
        z = 1.0 / (1.0 + jnp.exp(-z))
    o_ref[...] = z


def _combine(p, degp, h, ws, wn, b, sig):
    return pl.pallas_call(
        functools.partial(_combine_body, sig),
        grid=(NACC // BR,),
        in_specs=[
            pl.BlockSpec((2, BR, DH), lambda i: (0, i, 0)),
            pl.BlockSpec((2, BR), lambda i: (0, i)),
            pl.BlockSpec((BR, DH), lambda i: (i, 0)),
            pl.BlockSpec((DH, DH), lambda i: (0, 0)),
            pl.BlockSpec((DH, DH), lambda i: (0, 0)),
            pl.BlockSpec((1, DH), lambda i: (0, 0)),
        ],
        out_specs=pl.BlockSpec((BR, DH), lambda i: (i, 0)),
        out_shape=jax.ShapeDtypeStruct((NACC, DH), jnp.float32),
    )(p, degp, h, ws, wn, b)


def _readout_body(p_ref, degp_ref, h_ref, ws_ref, wn_ref, b_ref,
                  f_ref, wrf_ref, wrh_ref, bro_ref, o_ref):
    deg = jnp.sum(degp_ref[...], axis=0)
    recip = 1.0 / jnp.maximum(deg, 1.0)
    neigh = (p_ref[0] + p_ref[1]) * recip[:, None]
    z = (jnp.dot(h_ref[...], ws_ref[...], preferred_element_type=jnp.float32)
         + jnp.dot(neigh, wn_ref[...], preferred_element_type=jnp.float32)
         + b_ref[...])
    o_ref[...] = (jnp.dot(f_ref[...], wrf_ref[...],
                          preferred_element_type=jnp.float32)
                  + jnp.dot(z, wrh_ref[...],
                            preferred_element_type=jnp.float32)
                  + bro_ref[...])


def _readout(p, degp, h, ws, wn, b, feats, wrf, wrh, bro):
    return pl.pallas_call(
        _readout_body,
        grid=(NACC // BR,),
        in_specs=[
            pl.BlockSpec((2, BR, DH), lambda i: (0, i, 0)),
            pl.BlockSpec((2, BR), lambda i: (0, i)),
            pl.BlockSpec((BR, DH), lambda i: (i, 0)),
            pl.BlockSpec((DH, DH), lambda i: (0, 0)),
            pl.BlockSpec((DH, DH), lambda i: (0, 0)),
            pl.BlockSpec((1, DH), lambda i: (0, 0)),
            pl.BlockSpec((BR, D_IN), lambda i: (i, 0)),
            pl.BlockSpec((D_IN, 1), lambda i: (0, 0)),
            pl.BlockSpec((DH, 1), lambda i: (0, 0)),
            pl.BlockSpec((1, 1), lambda i: (0, 0)),
        ],
        out_specs=pl.BlockSpec((BR, 1), lambda i: (i, 0)),
        out_shape=jax.ShapeDtypeStruct((NACC, 1), jnp.float32),
    )(p, degp, h, ws, wn, b, feats, wrf, wrh, bro)


def kernel(features, edge_index, e_feat,
           W_self0, W_neigh0, b0,
           W_self1, W_neigh1, b1,
           W_self2, W_neigh2, b2,
           W_ro, b_ro):
    src = edge_index[0]
    dst = edge_index[1]
    pad = EPAD - E
    ar = jnp.arange(pad, dtype=jnp.int32)
    src_p = jnp.concatenate([src, ar % N])
    dst_p = jnp.concatenate([dst, N + ar % (NACC - N)])
    w_p = jnp.concatenate([e_feat[:, 0], jnp.zeros((pad,), jnp.float32)])

    h0 = jnp.concatenate([
        jnp.pad(features, ((0, NACC - N), (0, 0))),
        jnp.ones((NACC, 7), jnp.float32),
        jnp.zeros((NACC, DH - D_IN - 7), jnp.float32)], axis=1)
    z2d = jnp.zeros((NACC, DH), jnp.float32)
    Ws0 = jnp.pad(W_self0, ((0, DH - D_IN - 7), (0, 0)))
    Wn0 = jnp.pad(W_neigh0, ((0, DH - D_IN - 7), (0, 0)))

    z1d = jnp.zeros((NACC,), jnp.float32)
    acc0, degp = _edge_pass_deg(h0, src_p, dst_p, w_p, z2d, z1d)
    h1 = _combine(acc0, degp, h0, Ws0, Wn0, b0.reshape(1, DH), sig=True)
    acc1 = _edge_pass(h1, src_p, dst_p, w_p, z2d)
    h2 = _combine(acc1, degp, h1, W_self1, W_neigh1, b1.reshape(1, DH),
                  sig=True)
    acc2 = _edge_pass(h2, src_p, dst_p, w_p, z2d)
    feats_pad = jnp.pad(features, ((0, NACC - N), (0, 0)))
    out = _readout(acc2, degp, h2, W_self2, W_neigh2, b2.reshape(1, DH),
                   feats_pad, W_ro[:D_IN], W_ro[D_IN:], b_ro.reshape(1, 1))
    return out[:N]


# speedup vs baseline: 16.0220x; 2.5278x over previous
import functools

import jax
import jax.numpy as jnp
from jax import lax
from jax.experimental import pallas as pl
from jax.experimental.pallas import tpu as pltpu
from jax.experimental.pallas import tpu_sc as plsc

N = 100000
E = 1600000
D_IN = 3
DH = 16

NACC = 102400
C = 128
NW = 32
NCH = 402
EPAD = NW * C * NCH
EPT = EPAD // NW
RPT = NACC // 16
NR = 6


def _mk_edge_body(with_deg):
  def body(refs):
    if with_deg:
      (h_hbm, src_hbm, dst_hbm, w_hbm, z2_hbm, z1_hbm,
       acc_out, deg_out,
       srcv, dstv, wv, rows, ones_v, acc_sh, deg_sh,
       sgs, srs, sis, dg0, dg1) = refs
      dg = [dg0, dg1]
    else:
      (h_hbm, src_hbm, dst_hbm, w_hbm, z2_hbm,
       acc_out,
       srcv, dstv, wv, rows, acc_sh,
       sgs, srs, sis) = refs

    c = lax.axis_index("c")
    s = lax.axis_index("s")
    wid = c * 16 + s
    pltpu.sync_copy(z2_hbm.at[pl.ds(s * RPT, RPT)],
                    acc_sh.at[pl.ds(s * RPT, RPT)])
    if with_deg:
      pltpu.sync_copy(z1_hbm.at[pl.ds(s * RPT, RPT)],
                      deg_sh.at[pl.ds(s * RPT, RPT)])

      def _ob(j, carry):
        ones_v[pl.ds(j * 16, 16)] = jnp.ones((16,), jnp.float32)
        return carry
      lax.fori_loop(0, C // 16, _ob, 0)
    plsc.subcore_barrier()

    row0 = wid * NCH

    def idx_issue(cc, m):
      pltpu.async_copy(src_hbm.at[row0 + cc], srcv.at[m], sis.at[m])
      pltpu.async_copy(dst_hbm.at[row0 + cc], dstv.at[m], sis.at[m])
      pltpu.async_copy(w_hbm.at[row0 + cc], wv.at[m], sis.at[m])

    def idx_wait(cc, m):
      pltpu.make_async_copy(src_hbm.at[row0 + cc], srcv.at[m],
                            sis.at[m]).wait()
      pltpu.make_async_copy(dst_hbm.at[row0 + cc], dstv.at[m],
                            sis.at[m]).wait()
      pltpu.make_async_copy(w_hbm.at[row0 + cc], wv.at[m],
                            sis.at[m]).wait()

    def gather_issue(cc, m):
      pltpu.async_copy(h_hbm.at[srcv.at[m]], rows.at[m], sgs.at[m])

    def gather_wait(cc, m):
      pltpu.make_async_copy(h_hbm.at[srcv.at[m]], rows.at[m],
                            sgs.at[m]).wait()

    def scat_issue(cc, m):
      pltpu.async_copy(rows.at[m], acc_sh.at[dstv.at[m]], srs.at[m],
                       add=True)

    def scat_wait(cc, m):
      pltpu.make_async_copy(rows.at[m], acc_sh.at[dstv.at[m]],
                            srs.at[m]).wait()

    for m in range(4):
      idx_issue(m, m)
    for m in range(2):
      idx_wait(m, m)
      gather_issue(m, m)

    def _sextet(i, carry):
      for j in range(NR):
        cc = i * NR + j
        m = j
        gather_wait(cc, m)

        def _mul(g, carry2):
          wvec = wv[m, pl.ds(g * 16, 16)]
          for e in range(16):
            r = g * 16 + e
            rows[m, r, :] = rows[m, r, :] * wvec[e]
          return carry2
        lax.fori_loop(0, C // 16, _mul, 0)
        scat_issue(cc, m)
        if with_deg:
          q = j & 1
          deg_wait = pltpu.make_async_copy(
              ones_v, deg_sh.at[dstv.at[m]], dg[q]).wait
          if j < 2:
            pl.when(i >= 1)(deg_wait)
          else:
            deg_wait()
          pltpu.async_copy(ones_v, deg_sh.at[dstv.at[m]], dg[q], add=True)
        sw = functools.partial(scat_wait, cc - 2, (m + 4) % NR)
        if j < 2:
          pl.when(i >= 1)(sw)
        else:
          sw()
        def gi():
          idx_wait(cc + 2, (m + 2) % NR)
          gather_issue(cc + 2, (m + 2) % NR)
        if j < 4:
          gi()
        else:
          pl.when(i <= NCH // NR - 2)(gi)
        ii = functools.partial(idx_issue, cc + 4, (m + 4) % NR)
        if j < 2:
          ii()
        else:
          pl.when(i <= NCH // NR - 2)(ii)
      return carry
    lax.fori_loop(0, NCH // NR, _sextet, 0)
    scat_wait(NCH - 2, (NCH - 2) % NR)
    scat_wait(NCH - 1, (NCH - 1) % NR)
    if with_deg:
      pltpu.make_async_copy(ones_v, deg_sh.at[dstv.at[(NCH - 2) % NR]],
                            dg[0]).wait()
      pltpu.make_async_copy(ones_v, deg_sh.at[dstv.at[(NCH - 1) % NR]],
                            dg[1]).wait()
    plsc.subcore_barrier()
    pltpu.sync_copy(acc_sh.at[pl.ds(s * RPT, RPT)],
                    acc_out.at[c, pl.ds(s * RPT, RPT)])
    if with_deg:
      pltpu.sync_copy(deg_sh.at[pl.ds(s * RPT, RPT)],
                      deg_out.at[c, pl.ds(s * RPT, RPT)])
  return body


def _edge_pass_deg_body(*refs):
  return _mk_edge_body(True)(refs)


def _edge_pass_body(*refs):
  return _mk_edge_body(False)(refs)


_IDX_SCRATCH = lambda dt: pltpu.VMEM((RB, C), dt)
_ROWS_SCRATCH = lambda: pltpu.VMEM((C, DH), jnp.float32)


@functools.cache
def _build_edge_pass_deg():
  return functools.partial(
      pl.kernel,
      mesh=plsc.VectorSubcoreMesh(core_axis_name="c", subcore_axis_name="s"),
      compiler_params=pltpu.CompilerParams(use_tc_tiling_on_sc=False),
      out_type=(
          jax.ShapeDtypeStruct((2, NACC, DH), jnp.float32),
          jax.ShapeDtypeStruct((2, NACC), jnp.float32),
      ),
      scratch_types=[
          pltpu.VMEM((NR, C), jnp.int32),
          pltpu.VMEM((NR, C), jnp.int32),
          pltpu.VMEM((NR, C), jnp.float32),
          pltpu.VMEM((NR, C, DH), jnp.float32),
          pltpu.VMEM((C,), jnp.float32),
          pltpu.VMEM_SHARED((NACC, DH), jnp.float32),
          pltpu.VMEM_SHARED((NACC,), jnp.float32),
          pltpu.SemaphoreType.DMA((NR,)),
          pltpu.SemaphoreType.DMA((NR,)),
          pltpu.SemaphoreType.DMA((NR,)),
          pltpu.SemaphoreType.DMA,
          pltpu.SemaphoreType.DMA,
      ],
  )(_edge_pass_deg_body)


def _edge_pass_deg(*args):
  return _build_edge_pass_deg()(*args)


@functools.cache
def _build_edge_pass():
  return functools.partial(
      pl.kernel,
      mesh=plsc.VectorSubcoreMesh(core_axis_name="c", subcore_axis_name="s"),
      compiler_params=pltpu.CompilerParams(use_tc_tiling_on_sc=False),
      out_type=jax.ShapeDtypeStruct((2, NACC, DH), jnp.float32),
      scratch_types=[
          pltpu.VMEM((NR, C), jnp.int32),
          pltpu.VMEM((NR, C), jnp.int32),
          pltpu.VMEM((NR, C), jnp.float32),
          pltpu.VMEM((NR, C, DH), jnp.float32),
          pltpu.VMEM_SHARED((NACC, DH), jnp.float32),
          pltpu.SemaphoreType.DMA((NR,)),
          pltpu.SemaphoreType.DMA((NR,)),
          pltpu.SemaphoreType.DMA((NR,)),
      ],
  )(_edge_pass_body)


def _edge_pass(*args):
  return _build_edge_pass()(*args)


BR = 1024


def _combine_body(sig, p_ref, degp_ref, h_ref, ws_ref, wn_ref, b_ref, o_ref):
    deg = jnp.sum(degp_ref[...], axis=0)
    recip = 1.0 / jnp.maximum(deg, 1.0)
    neigh = (p_ref[0] + p_ref[1]) * recip[:, None]
    z = (jnp.dot(h_ref[...], ws_ref[...], preferred_element_type=jnp.float32)
         + jnp.dot(neigh, wn_ref[...], preferred_element_type=jnp.float32)
         + b_ref[...])
    if sig:
        z = 1.0 / (1.0 + jnp.exp(-z))
    o_ref[...] = z


def _combine(p, degp, h, ws, wn, b, sig):
    return pl.pallas_call(
        functools.partial(_combine_body, sig),
        grid=(NACC // BR,),
        in_specs=[
            pl.BlockSpec((2, BR, DH), lambda i: (0, i, 0)),
            pl.BlockSpec((2, BR), lambda i: (0, i)),
            pl.BlockSpec((BR, DH), lambda i: (i, 0)),
            pl.BlockSpec((DH, DH), lambda i: (0, 0)),
            pl.BlockSpec((DH, DH), lambda i: (0, 0)),
            pl.BlockSpec((1, DH), lambda i: (0, 0)),
        ],
        out_specs=pl.BlockSpec((BR, DH), lambda i: (i, 0)),
        out_shape=jax.ShapeDtypeStruct((NACC, DH), jnp.float32),
    )(p, degp, h, ws, wn, b)


def _readout_body(p_ref, degp_ref, h_ref, ws_ref, wn_ref, b_ref,
                  f_ref, wrf_ref, wrh_ref, bro_ref, o_ref):
    deg = jnp.sum(degp_ref[...], axis=0)
    recip = 1.0 / jnp.maximum(deg, 1.0)
    neigh = (p_ref[0] + p_ref[1]) * recip[:, None]
    z = (jnp.dot(h_ref[...], ws_ref[...], preferred_element_type=jnp.float32)
         + jnp.dot(neigh, wn_ref[...], preferred_element_type=jnp.float32)
         + b_ref[...])
    o_ref[...] = (jnp.dot(f_ref[...], wrf_ref[...],
                          preferred_element_type=jnp.float32)
                  + jnp.dot(z, wrh_ref[...],
                            preferred_element_type=jnp.float32)
                  + bro_ref[...])


def _readout(p, degp, h, ws, wn, b, feats, wrf, wrh, bro):
    return pl.pallas_call(
        _readout_body,
        grid=(NACC // BR,),
        in_specs=[
            pl.BlockSpec((2, BR, DH), lambda i: (0, i, 0)),
            pl.BlockSpec((2, BR), lambda i: (0, i)),
            pl.BlockSpec((BR, DH), lambda i: (i, 0)),
            pl.BlockSpec((DH, DH), lambda i: (0, 0)),
            pl.BlockSpec((DH, DH), lambda i: (0, 0)),
            pl.BlockSpec((1, DH), lambda i: (0, 0)),
            pl.BlockSpec((BR, D_IN), lambda i: (i, 0)),
            pl.BlockSpec((D_IN, 1), lambda i: (0, 0)),
            pl.BlockSpec((DH, 1), lambda i: (0, 0)),
            pl.BlockSpec((1, 1), lambda i: (0, 0)),
        ],
        out_specs=pl.BlockSpec((BR, 1), lambda i: (i, 0)),
        out_shape=jax.ShapeDtypeStruct((NACC, 1), jnp.float32),
    )(p, degp, h, ws, wn, b, feats, wrf, wrh, bro)


def kernel(features, edge_index, e_feat,
           W_self0, W_neigh0, b0,
           W_self1, W_neigh1, b1,
           W_self2, W_neigh2, b2,
           W_ro, b_ro):
    src = edge_index[0]
    dst = edge_index[1]
    pad = EPAD - E
    ar = jnp.arange(pad, dtype=jnp.int32)
    src_p = jnp.concatenate([src, ar % N]).reshape(EPAD // C, C)
    dst_p = jnp.concatenate([dst, N + ar % (NACC - N)]).reshape(EPAD // C, C)
    w_p = jnp.concatenate([e_feat[:, 0], jnp.zeros((pad,), jnp.float32)]
                          ).reshape(EPAD // C, C)

    h0 = jnp.concatenate([
        jnp.pad(features, ((0, NACC - N), (0, 0))),
        jnp.ones((NACC, 7), jnp.float32),
        jnp.zeros((NACC, DH - D_IN - 7), jnp.float32)], axis=1)
    z2d = jnp.zeros((NACC, DH), jnp.float32)
    Ws0 = jnp.pad(W_self0, ((0, DH - D_IN - 7), (0, 0)))
    Wn0 = jnp.pad(W_neigh0, ((0, DH - D_IN - 7), (0, 0)))

    z1d = jnp.zeros((NACC,), jnp.float32)
    acc0, degp = _edge_pass_deg(h0, src_p, dst_p, w_p, z2d, z1d)
    h1 = _combine(acc0, degp, h0, Ws0, Wn0, b0.reshape(1, DH), sig=True)
    acc1 = _edge_pass(h1, src_p, dst_p, w_p, z2d)
    h2 = _combine(acc1, degp, h1, W_self1, W_neigh1, b1.reshape(1, DH),
                  sig=True)
    acc2 = _edge_pass(h2, src_p, dst_p, w_p, z2d)
    feats_pad = jnp.pad(features, ((0, NACC - N), (0, 0)))
    out = _readout(acc2, degp, h2, W_self2, W_neigh2, b2.reshape(1, DH),
                   feats_pad, W_ro[:D_IN], W_ro[D_IN:], b_ro.reshape(1, 1))
    return out[:N]


# speedup vs baseline: 22.4959x; 1.4041x over previous
import functools

import jax
import jax.numpy as jnp
from jax import lax
from jax.experimental import pallas as pl
from jax.experimental.pallas import tpu as pltpu
from jax.experimental.pallas import tpu_sc as plsc

N = 100000
E = 1600000
D_IN = 3
DH = 16

NACC = 102400
C = 128
NW = 32
NCH = 402
EPAD = NW * C * NCH
EPT = EPAD // NW
RPT = NACC // 16
NR = 6
NF = NACC * DH // 128
NB = NF // 128


def _mk_edge_body(with_deg):
  def body(refs):
    if with_deg:
      (h_hbm, src_hbm, dst_hbm, w_hbm, z2_hbm, z1_hbm,
       acc_out, degx_out,
       srcv, dstv, wv, rows, ones_v, degc_v, degx_v, acc_sh, deg_sh,
       sgs, srs, sis, dg0, dg1) = refs
      dg = [dg0, dg1]
    else:
      (h_hbm, src_hbm, dst_hbm, w_hbm, z2_hbm,
       acc_out,
       srcv, dstv, wv, rows, acc_sh,
       sgs, srs, sis) = refs

    c = lax.axis_index("c")
    s = lax.axis_index("s")
    wid = c * 16 + s
    pltpu.sync_copy(z2_hbm.at[pl.ds(s * RPT, RPT)],
                    acc_sh.at[pl.ds(s * RPT, RPT)])
    if with_deg:
      pltpu.sync_copy(z1_hbm.at[pl.ds(s * RPT, RPT)],
                      deg_sh.at[pl.ds(s * RPT, RPT)])

      def _ob(j, carry):
        ones_v[pl.ds(j * 16, 16)] = jnp.ones((16,), jnp.float32)
        return carry
      lax.fori_loop(0, C // 16, _ob, 0)
    plsc.subcore_barrier()

    row0 = wid * NCH

    def idx_issue(cc, m):
      pltpu.async_copy(src_hbm.at[row0 + cc], srcv.at[m], sis.at[m])
      pltpu.async_copy(dst_hbm.at[row0 + cc], dstv.at[m], sis.at[m])
      pltpu.async_copy(w_hbm.at[row0 + cc], wv.at[m], sis.at[m])

    def idx_wait(cc, m):
      pltpu.make_async_copy(src_hbm.at[row0 + cc], srcv.at[m],
                            sis.at[m]).wait()
      pltpu.make_async_copy(dst_hbm.at[row0 + cc], dstv.at[m],
                            sis.at[m]).wait()
      pltpu.make_async_copy(w_hbm.at[row0 + cc], wv.at[m],
                            sis.at[m]).wait()

    def gather_issue(cc, m):
      pltpu.async_copy(h_hbm.at[srcv.at[m]], rows.at[m], sgs.at[m])

    def gather_wait(cc, m):
      pltpu.make_async_copy(h_hbm.at[srcv.at[m]], rows.at[m],
                            sgs.at[m]).wait()

    def scat_issue(cc, m):
      pltpu.async_copy(rows.at[m], acc_sh.at[dstv.at[m]], srs.at[m],
                       add=True)

    def scat_wait(cc, m):
      pltpu.make_async_copy(rows.at[m], acc_sh.at[dstv.at[m]],
                            srs.at[m]).wait()

    for m in range(4):
      idx_issue(m, m)
    for m in range(2):
      idx_wait(m, m)
      gather_issue(m, m)

    def _sextet(i, carry):
      for j in range(NR):
        cc = i * NR + j
        m = j
        gather_wait(cc, m)

        def _mul(g, carry2):
          wvec = wv[m, pl.ds(g * 16, 16)]
          for e in range(16):
            r = g * 16 + e
            rows[m, r, :] = rows[m, r, :] * wvec[e]
          return carry2
        lax.fori_loop(0, C // 16, _mul, 0)
        scat_issue(cc, m)
        if with_deg:
          q = j & 1
          deg_wait = pltpu.make_async_copy(
              ones_v, deg_sh.at[dstv.at[m]], dg[q]).wait
          if j < 2:
            pl.when(i >= 1)(deg_wait)
          else:
            deg_wait()
          pltpu.async_copy(ones_v, deg_sh.at[dstv.at[m]], dg[q], add=True)
        sw = functools.partial(scat_wait, cc - 2, (m + 4) % NR)
        if j < 2:
          pl.when(i >= 1)(sw)
        else:
          sw()
        def gi():
          idx_wait(cc + 2, (m + 2) % NR)
          gather_issue(cc + 2, (m + 2) % NR)
        if j < 4:
          gi()
        else:
          pl.when(i <= NCH // NR - 2)(gi)
        ii = functools.partial(idx_issue, cc + 4, (m + 4) % NR)
        if j < 2:
          ii()
        else:
          pl.when(i <= NCH // NR - 2)(ii)
      return carry
    lax.fori_loop(0, NCH // NR, _sextet, 0)
    scat_wait(NCH - 2, (NCH - 2) % NR)
    scat_wait(NCH - 1, (NCH - 1) % NR)
    if with_deg:
      pltpu.make_async_copy(ones_v, deg_sh.at[dstv.at[(NCH - 2) % NR]],
                            dg[0]).wait()
      pltpu.make_async_copy(ones_v, deg_sh.at[dstv.at[(NCH - 1) % NR]],
                            dg[1]).wait()
    plsc.subcore_barrier()
    pltpu.sync_copy(acc_sh.at[pl.ds(s * RPT, RPT)],
                    acc_out.at[c, pl.ds(s * RPT, RPT)])
    if with_deg:
      ones16 = jnp.ones((16,), jnp.float32)

      def _exp(t, carry):
        base = s * RPT + t * 256
        pltpu.sync_copy(deg_sh.at[pl.ds(base, 256)], degc_v)

        def _g(g, cc):
          dv = degc_v[pl.ds(g * 16, 16)]
          for e in range(16):
            degx_v[g * 16 + e, :] = ones16 * dv[e]
          return cc
        lax.fori_loop(0, 16, _g, 0)
        pltpu.sync_copy(degx_v, degx_out.at[c, pl.ds(base, 256)])
        return carry
      lax.fori_loop(0, RPT // 256, _exp, 0)
  return body


def _edge_pass_deg_body(*refs):
  return _mk_edge_body(True)(refs)


def _edge_pass_body(*refs):
  return _mk_edge_body(False)(refs)


_IDX_SCRATCH = lambda dt: pltpu.VMEM((RB, C), dt)
_ROWS_SCRATCH = lambda: pltpu.VMEM((C, DH), jnp.float32)


@functools.cache
def _build_edge_pass_deg():
  return functools.partial(
      pl.kernel,
      mesh=plsc.VectorSubcoreMesh(core_axis_name="c", subcore_axis_name="s"),
      compiler_params=pltpu.CompilerParams(use_tc_tiling_on_sc=False),
      out_type=(
          jax.ShapeDtypeStruct((2, NACC, DH), jnp.float32),
          jax.ShapeDtypeStruct((2, NACC, DH), jnp.float32),
      ),
      scratch_types=[
          pltpu.VMEM((NR, C), jnp.int32),
          pltpu.VMEM((NR, C), jnp.int32),
          pltpu.VMEM((NR, C), jnp.float32),
          pltpu.VMEM((NR, C, DH), jnp.float32),
          pltpu.VMEM((C,), jnp.float32),
          pltpu.VMEM((256,), jnp.float32),
          pltpu.VMEM((256, DH), jnp.float32),
          pltpu.VMEM_SHARED((NACC, DH), jnp.float32),
          pltpu.VMEM_SHARED((NACC,), jnp.float32),
          pltpu.SemaphoreType.DMA((NR,)),
          pltpu.SemaphoreType.DMA((NR,)),
          pltpu.SemaphoreType.DMA((NR,)),
          pltpu.SemaphoreType.DMA,
          pltpu.SemaphoreType.DMA,
      ],
  )(_edge_pass_deg_body)


def _edge_pass_deg(*args):
  return _build_edge_pass_deg()(*args)


@functools.cache
def _build_edge_pass():
  return functools.partial(
      pl.kernel,
      mesh=plsc.VectorSubcoreMesh(core_axis_name="c", subcore_axis_name="s"),
      compiler_params=pltpu.CompilerParams(use_tc_tiling_on_sc=False),
      out_type=jax.ShapeDtypeStruct((2, NACC, DH), jnp.float32),
      scratch_types=[
          pltpu.VMEM((NR, C), jnp.int32),
          pltpu.VMEM((NR, C), jnp.int32),
          pltpu.VMEM((NR, C), jnp.float32),
          pltpu.VMEM((NR, C, DH), jnp.float32),
          pltpu.VMEM_SHARED((NACC, DH), jnp.float32),
          pltpu.SemaphoreType.DMA((NR,)),
          pltpu.SemaphoreType.DMA((NR,)),
          pltpu.SemaphoreType.DMA((NR,)),
      ],
  )(_edge_pass_body)


def _edge_pass(*args):
  return _build_edge_pass()(*args)


def _combine_body(sig, p0_ref, p1_ref, d0_ref, d1_ref, h_ref, bs_ref,
                  bn_ref, bp_ref, o_ref):
    rx = 1.0 / jnp.maximum(d0_ref[...] + d1_ref[...], 1.0)
    neigh = (p0_ref[...] + p1_ref[...]) * rx
    z = (jnp.dot(h_ref[...], bs_ref[...], preferred_element_type=jnp.float32)
         + jnp.dot(neigh, bn_ref[...], preferred_element_type=jnp.float32)
         + bp_ref[...])
    if sig:
        z = 1.0 / (1.0 + jnp.exp(-z))
    o_ref[...] = z


_B128 = lambda: pl.BlockSpec((128, 128), lambda i: (i, 0))
_B128H = lambda: pl.BlockSpec((128, 128), lambda i: (i + NB, 0))
_BW = lambda: pl.BlockSpec((128, 128), lambda i: (0, 0))


def _combine(accf, degxf, hf, bs, bn, bp, sig):
    return pl.pallas_call(
        functools.partial(_combine_body, sig),
        grid=(NB,),
        in_specs=[
            _B128(), _B128H(), _B128(), _B128H(), _B128(),
            _BW(), _BW(),
            pl.BlockSpec((1, 128), lambda i: (0, 0)),
        ],
        out_specs=pl.BlockSpec((128, 128), lambda i: (i, 0)),
        out_shape=jax.ShapeDtypeStruct((NF, 128), jnp.float32),
    )(accf, accf, degxf, degxf, hf, bs, bn, bp)


def _readout_body(p0_ref, p1_ref, d0_ref, d1_ref, h_ref, h0_ref, bs_ref,
                  bn_ref, bp_ref, kro_ref, kf_ref, o_ref):
    rx = 1.0 / jnp.maximum(d0_ref[...] + d1_ref[...], 1.0)
    neigh = (p0_ref[...] + p1_ref[...]) * rx
    z = (jnp.dot(h_ref[...], bs_ref[...], preferred_element_type=jnp.float32)
         + jnp.dot(neigh, bn_ref[...], preferred_element_type=jnp.float32)
         + bp_ref[...])
    o_ref[...] = (jnp.dot(z, kro_ref[...], preferred_element_type=jnp.float32)
                  + jnp.dot(h0_ref[...], kf_ref[...],
                            preferred_element_type=jnp.float32))


def _readout(accf, degxf, hf, h0f, bs, bn, bp, kro, kf):
    return pl.pallas_call(
        _readout_body,
        grid=(NB,),
        in_specs=[
            _B128(), _B128H(), _B128(), _B128H(), _B128(), _B128(),
            _BW(), _BW(),
            pl.BlockSpec((1, 128), lambda i: (0, 0)),
            _BW(), _BW(),
        ],
        out_specs=pl.BlockSpec((128, 128), lambda i: (i, 0)),
        out_shape=jax.ShapeDtypeStruct((NF, 128), jnp.float32),
    )(accf, accf, degxf, degxf, hf, h0f, bs, bn, bp, kro, kf)


def _pack16(w):
    return jnp.kron(jnp.eye(8, dtype=jnp.float32), w)


def kernel(features, edge_index, e_feat,
           W_self0, W_neigh0, b0,
           W_self1, W_neigh1, b1,
           W_self2, W_neigh2, b2,
           W_ro, b_ro):
    src = edge_index[0]
    dst = edge_index[1]
    pad = EPAD - E
    ar = jnp.arange(pad, dtype=jnp.int32)
    src_p = jnp.concatenate([src, ar % N]).reshape(EPAD // C, C)
    dst_p = jnp.concatenate([dst, N + ar % (NACC - N)]).reshape(EPAD // C, C)
    w_p = jnp.concatenate([e_feat[:, 0], jnp.zeros((pad,), jnp.float32)]
                          ).reshape(EPAD // C, C)

    z2d = jnp.zeros((NACC, DH), jnp.float32)
    z1d = jnp.zeros((NACC,), jnp.float32)
    h0 = jnp.concatenate([
        jnp.pad(features, ((0, NACC - N), (0, 0))),
        jnp.ones((NACC, 7), jnp.float32),
        jnp.zeros((NACC, DH - D_IN - 7), jnp.float32)], axis=1)
    h0f = h0.reshape(NF, 128)

    Ws0 = jnp.pad(W_self0, ((0, DH - D_IN - 7), (0, 0)))
    Wn0 = jnp.pad(W_neigh0, ((0, DH - D_IN - 7), (0, 0)))
    bpack = lambda b: jnp.tile(b, 8).reshape(1, 128)
    ones16r = jnp.ones((1, DH), jnp.float32)
    wfx = jnp.concatenate(
        [W_ro[:D_IN], b_ro.reshape(1, 1),
         jnp.zeros((DH - D_IN - 1, 1), jnp.float32)], axis=0)
    kro = _pack16(W_ro[D_IN:] @ ones16r)
    kf = _pack16(wfx @ ones16r)

    flat = lambda a: a.reshape(2 * NF, 128)
    acc0, degx = _edge_pass_deg(h0, src_p, dst_p, w_p, z2d, z1d)
    degxf = flat(degx)
    h1f = _combine(flat(acc0), degxf, h0f, _pack16(Ws0), _pack16(Wn0),
                   bpack(b0), sig=True)
    acc1 = _edge_pass(h1f.reshape(NACC, DH), src_p, dst_p, w_p, z2d)
    h2f = _combine(flat(acc1), degxf, h1f, _pack16(W_self1),
                   _pack16(W_neigh1), bpack(b1), sig=True)
    acc2 = _edge_pass(h2f.reshape(NACC, DH), src_p, dst_p, w_p, z2d)
    outf = _readout(flat(acc2), degxf, h2f, h0f, _pack16(W_self2),
                    _pack16(W_neigh2), bpack(b2), kro, kf)
    return outf.reshape(NACC, DH)[:N, :1]


# speedup vs baseline: 26.6724x; 1.1857x over previous
import functools

import jax
import jax.numpy as jnp
from jax import lax
from jax.experimental import pallas as pl
from jax.experimental.pallas import tpu as pltpu
from jax.experimental.pallas import tpu_sc as plsc

N = 100000
E = 1600000
D_IN = 3
DH = 16

NACC = 102400
C = 128
NW = 32
NCH = 400
EPAD = NW * C * NCH
EPT = EPAD // NW
RPT = NACC // 16
NR = 8
NF = NACC * DH // 128
NB = NF // 128


def _mk_edge_body(with_deg):
  def body(refs):
    if with_deg:
      (h_hbm, src_hbm, dst_hbm, w_hbm, z2_hbm, z1_hbm,
       acc_out, degx_out,
       srcv, dstv, wv, rows, ones_v, degc_v, degx_v, acc_sh, deg_sh,
       sgs, srs, sis, dg0, dg1) = refs
      dg = [dg0, dg1]
    else:
      (h_hbm, src_hbm, dst_hbm, w_hbm, z2_hbm,
       acc_out,
       srcv, dstv, wv, rows, acc_sh,
       sgs, srs, sis) = refs

    c = lax.axis_index("c")
    s = lax.axis_index("s")
    wid = c * 16 + s
    pltpu.sync_copy(z2_hbm.at[pl.ds(s * RPT, RPT)],
                    acc_sh.at[pl.ds(s * RPT, RPT)])
    if with_deg:
      pltpu.sync_copy(z1_hbm.at[pl.ds(s * RPT, RPT)],
                      deg_sh.at[pl.ds(s * RPT, RPT)])

      def _ob(j, carry):
        ones_v[pl.ds(j * 16, 16)] = jnp.ones((16,), jnp.float32)
        return carry
      lax.fori_loop(0, C // 16, _ob, 0)
    plsc.subcore_barrier()

    row0 = wid * NCH

    def idx_issue(cc, m):
      pltpu.async_copy(src_hbm.at[row0 + cc], srcv.at[m], sis.at[m])
      pltpu.async_copy(dst_hbm.at[row0 + cc], dstv.at[m], sis.at[m])
      pltpu.async_copy(w_hbm.at[row0 + cc], wv.at[m], sis.at[m])

    def idx_wait(cc, m):
      pltpu.make_async_copy(src_hbm.at[row0 + cc], srcv.at[m],
                            sis.at[m]).wait()
      pltpu.make_async_copy(dst_hbm.at[row0 + cc], dstv.at[m],
                            sis.at[m]).wait()
      pltpu.make_async_copy(w_hbm.at[row0 + cc], wv.at[m],
                            sis.at[m]).wait()

    def gather_issue(cc, m):
      pltpu.async_copy(h_hbm.at[srcv.at[m]], rows.at[m], sgs.at[m])

    def gather_wait(cc, m):
      pltpu.make_async_copy(h_hbm.at[srcv.at[m]], rows.at[m],
                            sgs.at[m]).wait()

    def scat_issue(cc, m):
      pltpu.async_copy(rows.at[m], acc_sh.at[dstv.at[m]], srs.at[m],
                       add=True)

    def scat_wait(cc, m):
      pltpu.make_async_copy(rows.at[m], acc_sh.at[dstv.at[m]],
                            srs.at[m]).wait()

    for m in range(6):
      idx_issue(m, m)
    for m in range(4):
      idx_wait(m, m)
      gather_issue(m, m)

    def _octet(i, carry):
      for j in range(NR):
        cc = i * NR + j
        m = j
        gather_wait(cc, m)

        def _mul(g, carry2):
          wvec = wv[m, pl.ds(g * 16, 16)]
          for e in range(16):
            r = g * 16 + e
            rows[m, r, :] = rows[m, r, :] * wvec[e]
          return carry2
        lax.fori_loop(0, C // 16, _mul, 0)
        scat_issue(cc, m)
        if with_deg:
          q = j & 1
          deg_wait = pltpu.make_async_copy(
              ones_v, deg_sh.at[dstv.at[m]], dg[q]).wait
          if j < 2:
            pl.when(i >= 1)(deg_wait)
          else:
            deg_wait()
          pltpu.async_copy(ones_v, deg_sh.at[dstv.at[m]], dg[q], add=True)
        sw = functools.partial(scat_wait, cc - 2, (m + 6) % NR)
        if j < 2:
          pl.when(i >= 1)(sw)
        else:
          sw()
        ii = functools.partial(idx_issue, cc + 6, (m + 6) % NR)
        if j < 2:
          ii()
        else:
          pl.when(i <= NCH // NR - 2)(ii)
        def gi():
          idx_wait(cc + 4, (m + 4) % NR)
          gather_issue(cc + 4, (m + 4) % NR)
        if j < 4:
          gi()
        else:
          pl.when(i <= NCH // NR - 2)(gi)
      return carry
    lax.fori_loop(0, NCH // NR, _octet, 0)
    scat_wait(NCH - 2, (NCH - 2) % NR)
    scat_wait(NCH - 1, (NCH - 1) % NR)
    if with_deg:
      pltpu.make_async_copy(ones_v, deg_sh.at[dstv.at[(NCH - 2) % NR]],
                            dg[0]).wait()
      pltpu.make_async_copy(ones_v, deg_sh.at[dstv.at[(NCH - 1) % NR]],
                            dg[1]).wait()
    plsc.subcore_barrier()
    pltpu.sync_copy(acc_sh.at[pl.ds(s * RPT, RPT)],
                    acc_out.at[c, pl.ds(s * RPT, RPT)])
    if with_deg:
      ones16 = jnp.ones((16,), jnp.float32)

      def _exp(t, carry):
        base = s * RPT + t * 128
        pltpu.sync_copy(deg_sh.at[pl.ds(base, 128)], degc_v)

        def _g(g, cc):
          dv = degc_v[pl.ds(g * 16, 16)]
          for e in range(16):
            degx_v[g * 16 + e, :] = ones16 * dv[e]
          return cc
        lax.fori_loop(0, 8, _g, 0)
        pltpu.sync_copy(degx_v, degx_out.at[c, pl.ds(base, 128)])
        return carry
      lax.fori_loop(0, RPT // 128, _exp, 0)
  return body


def _edge_pass_deg_body(*refs):
  return _mk_edge_body(True)(refs)


def _edge_pass_body(*refs):
  return _mk_edge_body(False)(refs)


_IDX_SCRATCH = lambda dt: pltpu.VMEM((RB, C), dt)
_ROWS_SCRATCH = lambda: pltpu.VMEM((C, DH), jnp.float32)


@functools.cache
def _build_edge_pass_deg():
  return functools.partial(
      pl.kernel,
      mesh=plsc.VectorSubcoreMesh(core_axis_name="c", subcore_axis_name="s"),
      compiler_params=pltpu.CompilerParams(use_tc_tiling_on_sc=False),
      out_type=(
          jax.ShapeDtypeStruct((2, NACC, DH), jnp.float32),
          jax.ShapeDtypeStruct((2, NACC, DH), jnp.float32),
      ),
      scratch_types=[
          pltpu.VMEM((NR, C), jnp.int32),
          pltpu.VMEM((NR, C), jnp.int32),
          pltpu.VMEM((NR, C), jnp.float32),
          pltpu.VMEM((NR, C, DH), jnp.float32),
          pltpu.VMEM((C,), jnp.float32),
          pltpu.VMEM((128,), jnp.float32),
          pltpu.VMEM((128, DH), jnp.float32),
          pltpu.VMEM_SHARED((NACC, DH), jnp.float32),
          pltpu.VMEM_SHARED((NACC,), jnp.float32),
          pltpu.SemaphoreType.DMA((NR,)),
          pltpu.SemaphoreType.DMA((NR,)),
          pltpu.SemaphoreType.DMA((NR,)),
          pltpu.SemaphoreType.DMA,
          pltpu.SemaphoreType.DMA,
      ],
  )(_edge_pass_deg_body)


def _edge_pass_deg(*args):
  return _build_edge_pass_deg()(*args)


@functools.cache
def _build_edge_pass():
  return functools.partial(
      pl.kernel,
      mesh=plsc.VectorSubcoreMesh(core_axis_name="c", subcore_axis_name="s"),
      compiler_params=pltpu.CompilerParams(use_tc_tiling_on_sc=False),
      out_type=jax.ShapeDtypeStruct((2, NACC, DH), jnp.float32),
      scratch_types=[
          pltpu.VMEM((NR, C), jnp.int32),
          pltpu.VMEM((NR, C), jnp.int32),
          pltpu.VMEM((NR, C), jnp.float32),
          pltpu.VMEM((NR, C, DH), jnp.float32),
          pltpu.VMEM_SHARED((NACC, DH), jnp.float32),
          pltpu.SemaphoreType.DMA((NR,)),
          pltpu.SemaphoreType.DMA((NR,)),
          pltpu.SemaphoreType.DMA((NR,)),
      ],
  )(_edge_pass_body)


def _edge_pass(*args):
  return _build_edge_pass()(*args)


def _rx_body(d0_ref, d1_ref, o_ref):
    o_ref[...] = 1.0 / jnp.maximum(d0_ref[...] + d1_ref[...], 1.0)


def _rx(degxf):
    return pl.pallas_call(
        _rx_body,
        grid=(NB,),
        in_specs=[
            pl.BlockSpec((128, 128), lambda i: (i, 0)),
            pl.BlockSpec((128, 128), lambda i: (i + NB, 0)),
        ],
        out_specs=pl.BlockSpec((128, 128), lambda i: (i, 0)),
        out_shape=jax.ShapeDtypeStruct((NF, 128), jnp.float32),
    )(degxf, degxf)


def _combine_body(sig, p0_ref, p1_ref, rx_ref, h_ref, bs_ref,
                  bn_ref, bp_ref, o_ref):
    neigh = (p0_ref[...] + p1_ref[...]) * rx_ref[...]
    z = (jnp.dot(h_ref[...], bs_ref[...], preferred_element_type=jnp.float32)
         + jnp.dot(neigh, bn_ref[...], preferred_element_type=jnp.float32)
         + bp_ref[...])
    if sig:
        z = 1.0 / (1.0 + jnp.exp(-z))
    o_ref[...] = z


_B128 = lambda: pl.BlockSpec((128, 128), lambda i: (i, 0))
_B128H = lambda: pl.BlockSpec((128, 128), lambda i: (i + NB, 0))
_BW = lambda: pl.BlockSpec((128, 128), lambda i: (0, 0))


def _combine(accf, rxf, hf, bs, bn, bp, sig):
    return pl.pallas_call(
        functools.partial(_combine_body, sig),
        grid=(NB,),
        in_specs=[
            _B128(), _B128H(), _B128(), _B128(),
            _BW(), _BW(),
            pl.BlockSpec((1, 128), lambda i: (0, 0)),
        ],
        out_specs=pl.BlockSpec((128, 128), lambda i: (i, 0)),
        out_shape=jax.ShapeDtypeStruct((NF, 128), jnp.float32),
    )(accf, accf, rxf, hf, bs, bn, bp)


def _readout_body(p0_ref, p1_ref, rx_ref, h_ref, h0_ref, bs_ref,
                  bn_ref, bp_ref, kro_ref, kf_ref, o_ref):
    neigh = (p0_ref[...] + p1_ref[...]) * rx_ref[...]
    z = (jnp.dot(h_ref[...], bs_ref[...], preferred_element_type=jnp.float32)
         + jnp.dot(neigh, bn_ref[...], preferred_element_type=jnp.float32)
         + bp_ref[...])
    o_ref[...] = (jnp.dot(z, kro_ref[...], preferred_element_type=jnp.float32)
                  + jnp.dot(h0_ref[...], kf_ref[...],
                            preferred_element_type=jnp.float32))


def _readout(accf, rxf, hf, h0f, bs, bn, bp, kro, kf):
    return pl.pallas_call(
        _readout_body,
        grid=(NB,),
        in_specs=[
            _B128(), _B128H(), _B128(), _B128(), _B128(),
            _BW(), _BW(),
            pl.BlockSpec((1, 128), lambda i: (0, 0)),
            _BW(), _BW(),
        ],
        out_specs=pl.BlockSpec((128, 128), lambda i: (i, 0)),
        out_shape=jax.ShapeDtypeStruct((NF, 128), jnp.float32),
    )(accf, accf, rxf, hf, h0f, bs, bn, bp, kro, kf)


def _pack16(w):
    return jnp.kron(jnp.eye(8, dtype=jnp.float32), w)


def kernel(features, edge_index, e_feat,
           W_self0, W_neigh0, b0,
           W_self1, W_neigh1, b1,
           W_self2, W_neigh2, b2,
           W_ro, b_ro):
    src = edge_index[0]
    dst = edge_index[1]
    pad = EPAD - E
    ar = jnp.arange(pad, dtype=jnp.int32)
    src_p = jnp.concatenate([src, ar % N]).reshape(EPAD // C, C)
    dst_p = jnp.concatenate([dst, N + ar % (NACC - N)]).reshape(EPAD // C, C)
    w_p = jnp.concatenate([e_feat[:, 0], jnp.zeros((pad,), jnp.float32)]
                          ).reshape(EPAD // C, C)

    z2d = jnp.zeros((NACC, DH), jnp.float32)
    z1d = jnp.zeros((NACC,), jnp.float32)
    h0 = jnp.concatenate([
        jnp.pad(features, ((0, NACC - N), (0, 0))),
        jnp.ones((NACC, 7), jnp.float32),
        jnp.zeros((NACC, DH - D_IN - 7), jnp.float32)], axis=1)
    h0f = h0.reshape(NF, 128)

    Ws0 = jnp.pad(W_self0, ((0, DH - D_IN - 7), (0, 0)))
    Wn0 = jnp.pad(W_neigh0, ((0, DH - D_IN - 7), (0, 0)))
    bpack = lambda b: jnp.tile(b, 8).reshape(1, 128)
    ones16r = jnp.ones((1, DH), jnp.float32)
    wfx = jnp.concatenate(
        [W_ro[:D_IN], b_ro.reshape(1, 1),
         jnp.zeros((DH - D_IN - 1, 1), jnp.float32)], axis=0)
    kro = _pack16(W_ro[D_IN:] @ ones16r)
    kf = _pack16(wfx @ ones16r)

    flat = lambda a: a.reshape(2 * NF, 128)
    acc0, degx = _edge_pass_deg(h0, src_p, dst_p, w_p, z2d, z1d)
    rxf = _rx(flat(degx))
    h1f = _combine(flat(acc0), rxf, h0f, _pack16(Ws0), _pack16(Wn0),
                   bpack(b0), sig=True)
    acc1 = _edge_pass(h1f.reshape(NACC, DH), src_p, dst_p, w_p, z2d)
    h2f = _combine(flat(acc1), rxf, h1f, _pack16(W_self1),
                   _pack16(W_neigh1), bpack(b1), sig=True)
    acc2 = _edge_pass(h2f.reshape(NACC, DH), src_p, dst_p, w_p, z2d)
    outf = _readout(flat(acc2), rxf, h2f, h0f, _pack16(W_self2),
                    _pack16(W_neigh2), bpack(b2), kro, kf)
    return outf.reshape(NACC, DH)[:N, :1]


# speedup vs baseline: 26.6755x; 1.0001x over previous
import functools

import jax
import jax.numpy as jnp
from jax import lax
from jax.experimental import pallas as pl
from jax.experimental.pallas import tpu as pltpu
from jax.experimental.pallas import tpu_sc as plsc

N = 100000
E = 1600000
D_IN = 3
DH = 16

NACC = 102400
C = 128
NW = 32
NCH = 400
EPAD = NW * C * NCH
EPT = EPAD // NW
RPT = NACC // 16
NR = 8
NF = NACC * DH // 128
NB = NF // 128


def _mk_edge_body(with_deg):
  def body(refs):
    if with_deg:
      (h_hbm, src_hbm, dst_hbm, w_hbm, z2_hbm, z1_hbm,
       acc_out, degx_out, hcopy_out,
       srcv, dstv, wv, rows, ones_v, degc_v, degx_v, acc_sh, deg_sh,
       sgs, srs, sis, dg0, dg1) = refs
      dg = [dg0, dg1]
    else:
      (h_hbm, src_hbm, dst_hbm, w_hbm, z2_hbm,
       acc_out,
       srcv, dstv, wv, rows, acc_sh,
       sgs, srs, sis) = refs

    c = lax.axis_index("c")
    s = lax.axis_index("s")
    wid = c * 16 + s
    pltpu.sync_copy(z2_hbm.at[pl.ds(s * RPT, RPT)],
                    acc_sh.at[pl.ds(s * RPT, RPT)])
    if with_deg:
      pltpu.sync_copy(z1_hbm.at[pl.ds(s * RPT, RPT)],
                      deg_sh.at[pl.ds(s * RPT, RPT)])
      pltpu.sync_copy(h_hbm.at[pl.ds(wid * (NACC // NW), NACC // NW)],
                      hcopy_out.at[pl.ds(wid * (NACC // NW), NACC // NW)])

      def _ob(j, carry):
        ones_v[pl.ds(j * 16, 16)] = jnp.ones((16,), jnp.float32)
        return carry
      lax.fori_loop(0, C // 16, _ob, 0)
    plsc.subcore_barrier()

    row0 = wid * NCH

    def idx_issue(cc, m):
      pltpu.async_copy(src_hbm.at[row0 + cc], srcv.at[m], sis.at[m])
      pltpu.async_copy(dst_hbm.at[row0 + cc], dstv.at[m], sis.at[m])
      pltpu.async_copy(w_hbm.at[row0 + cc], wv.at[m], sis.at[m])

    def idx_wait(cc, m):
      pltpu.make_async_copy(src_hbm.at[row0 + cc], srcv.at[m],
                            sis.at[m]).wait()
      pltpu.make_async_copy(dst_hbm.at[row0 + cc], dstv.at[m],
                            sis.at[m]).wait()
      pltpu.make_async_copy(w_hbm.at[row0 + cc], wv.at[m],
                            sis.at[m]).wait()

    def gather_issue(cc, m):
      pltpu.async_copy(h_hbm.at[srcv.at[m]], rows.at[m], sgs.at[m])

    def gather_wait(cc, m):
      pltpu.make_async_copy(h_hbm.at[srcv.at[m]], rows.at[m],
                            sgs.at[m]).wait()

    def scat_issue(cc, m):
      pltpu.async_copy(rows.at[m], acc_sh.at[dstv.at[m]], srs.at[m],
                       add=True)

    def scat_wait(cc, m):
      pltpu.make_async_copy(rows.at[m], acc_sh.at[dstv.at[m]],
                            srs.at[m]).wait()

    for m in range(6):
      idx_issue(m, m)
    for m in range(4):
      idx_wait(m, m)
      gather_issue(m, m)

    def _octet(i, carry):
      for j in range(NR):
        cc = i * NR + j
        m = j
        gather_wait(cc, m)

        def _mul(g, carry2):
          wvec = wv[m, pl.ds(g * 16, 16)]
          for e in range(16):
            r = g * 16 + e
            rows[m, r, :] = rows[m, r, :] * wvec[e]
          return carry2
        lax.fori_loop(0, C // 16, _mul, 0)
        scat_issue(cc, m)
        if with_deg:
          q = j & 1
          deg_wait = pltpu.make_async_copy(
              ones_v, deg_sh.at[dstv.at[m]], dg[q]).wait
          if j < 2:
            pl.when(i >= 1)(deg_wait)
          else:
            deg_wait()
          pltpu.async_copy(ones_v, deg_sh.at[dstv.at[m]], dg[q], add=True)
        sw = functools.partial(scat_wait, cc - 2, (m + 6) % NR)
        if j < 2:
          pl.when(i >= 1)(sw)
        else:
          sw()
        ii = functools.partial(idx_issue, cc + 6, (m + 6) % NR)
        if j < 2:
          ii()
        else:
          pl.when(i <= NCH // NR - 2)(ii)
        def gi():
          idx_wait(cc + 4, (m + 4) % NR)
          gather_issue(cc + 4, (m + 4) % NR)
        if j < 4:
          gi()
        else:
          pl.when(i <= NCH // NR - 2)(gi)
      return carry
    lax.fori_loop(0, NCH // NR, _octet, 0)
    scat_wait(NCH - 2, (NCH - 2) % NR)
    scat_wait(NCH - 1, (NCH - 1) % NR)
    if with_deg:
      pltpu.make_async_copy(ones_v, deg_sh.at[dstv.at[(NCH - 2) % NR]],
                            dg[0]).wait()
      pltpu.make_async_copy(ones_v, deg_sh.at[dstv.at[(NCH - 1) % NR]],
                            dg[1]).wait()
    plsc.subcore_barrier()
    pltpu.sync_copy(acc_sh.at[pl.ds(s * RPT, RPT)],
                    acc_out.at[c, pl.ds(s * RPT, RPT)])
    if with_deg:
      ones16 = jnp.ones((16,), jnp.float32)

      def _exp(t, carry):
        base = s * RPT + t * 128
        pltpu.sync_copy(deg_sh.at[pl.ds(base, 128)], degc_v)

        def _g(g, cc):
          dv = degc_v[pl.ds(g * 16, 16)]
          for e in range(16):
            degx_v[g * 16 + e, :] = ones16 * dv[e]
          return cc
        lax.fori_loop(0, 8, _g, 0)
        pltpu.sync_copy(degx_v, degx_out.at[c, pl.ds(base, 128)])
        return carry
      lax.fori_loop(0, RPT // 128, _exp, 0)
  return body


def _edge_pass_deg_body(*refs):
  return _mk_edge_body(True)(refs)


def _edge_pass_body(*refs):
  return _mk_edge_body(False)(refs)


_IDX_SCRATCH = lambda dt: pltpu.VMEM((RB, C), dt)
_ROWS_SCRATCH = lambda: pltpu.VMEM((C, DH), jnp.float32)


@functools.cache
def _build_edge_pass_deg():
  return functools.partial(
      pl.kernel,
      mesh=plsc.VectorSubcoreMesh(core_axis_name="c", subcore_axis_name="s"),
      compiler_params=pltpu.CompilerParams(use_tc_tiling_on_sc=False),
      out_type=(
          jax.ShapeDtypeStruct((2, NACC, DH), jnp.float32),
          jax.ShapeDtypeStruct((2, NACC, DH), jnp.float32),
          jax.ShapeDtypeStruct((NACC, DH), jnp.float32),
      ),
      scratch_types=[
          pltpu.VMEM((NR, C), jnp.int32),
          pltpu.VMEM((NR, C), jnp.int32),
          pltpu.VMEM((NR, C), jnp.float32),
          pltpu.VMEM((NR, C, DH), jnp.float32),
          pltpu.VMEM((C,), jnp.float32),
          pltpu.VMEM((128,), jnp.float32),
          pltpu.VMEM((128, DH), jnp.float32),
          pltpu.VMEM_SHARED((NACC, DH), jnp.float32),
          pltpu.VMEM_SHARED((NACC,), jnp.float32),
          pltpu.SemaphoreType.DMA((NR,)),
          pltpu.SemaphoreType.DMA((NR,)),
          pltpu.SemaphoreType.DMA((NR,)),
          pltpu.SemaphoreType.DMA,
          pltpu.SemaphoreType.DMA,
      ],
  )(_edge_pass_deg_body)


def _edge_pass_deg(*args):
  return _build_edge_pass_deg()(*args)


@functools.cache
def _build_edge_pass():
  return functools.partial(
      pl.kernel,
      mesh=plsc.VectorSubcoreMesh(core_axis_name="c", subcore_axis_name="s"),
      compiler_params=pltpu.CompilerParams(use_tc_tiling_on_sc=False),
      out_type=jax.ShapeDtypeStruct((2, NACC, DH), jnp.float32),
      scratch_types=[
          pltpu.VMEM((NR, C), jnp.int32),
          pltpu.VMEM((NR, C), jnp.int32),
          pltpu.VMEM((NR, C), jnp.float32),
          pltpu.VMEM((NR, C, DH), jnp.float32),
          pltpu.VMEM_SHARED((NACC, DH), jnp.float32),
          pltpu.SemaphoreType.DMA((NR,)),
          pltpu.SemaphoreType.DMA((NR,)),
          pltpu.SemaphoreType.DMA((NR,)),
      ],
  )(_edge_pass_body)


def _edge_pass(*args):
  return _build_edge_pass()(*args)


def _rx_body(d0_ref, d1_ref, o_ref):
    o_ref[...] = 1.0 / jnp.maximum(d0_ref[...] + d1_ref[...], 1.0)


def _rx(degxf):
    return pl.pallas_call(
        _rx_body,
        grid=(NGB,),
        in_specs=[_B128(), _B128H()],
        out_specs=pl.BlockSpec((BH, 128), lambda i: (i, 0)),
        out_shape=jax.ShapeDtypeStruct((NF, 128), jnp.float32),
    )(degxf, degxf)


def _combine_body(sig, p0_ref, p1_ref, rx_ref, h_ref, bs_ref,
                  bn_ref, bp_ref, o_ref):
    neigh = (p0_ref[...] + p1_ref[...]) * rx_ref[...]
    z = (jnp.dot(h_ref[...], bs_ref[...], preferred_element_type=jnp.float32)
         + jnp.dot(neigh, bn_ref[...], preferred_element_type=jnp.float32)
         + bp_ref[...])
    if sig:
        z = 1.0 / (1.0 + jnp.exp(-z))
    o_ref[...] = z


BH = 1280
NGB = NF // BH
_B128 = lambda: pl.BlockSpec((BH, 128), lambda i: (i, 0))
_B128H = lambda: pl.BlockSpec((BH, 128), lambda i: (i + NGB, 0))
_BW = lambda: pl.BlockSpec((128, 128), lambda i: (0, 0))


def _combine(accf, rxf, hf, bs, bn, bp, sig):
    return pl.pallas_call(
        functools.partial(_combine_body, sig),
        grid=(NGB,),
        in_specs=[
            _B128(), _B128H(), _B128(), _B128(),
            _BW(), _BW(),
            pl.BlockSpec((1, 128), lambda i: (0, 0)),
        ],
        out_specs=pl.BlockSpec((BH, 128), lambda i: (i, 0)),
        out_shape=jax.ShapeDtypeStruct((NF, 128), jnp.float32),
    )(accf, accf, rxf, hf, bs, bn, bp)


def _readout_body(p0_ref, p1_ref, rx_ref, h_ref, h0_ref, bs_ref,
                  bn_ref, bp_ref, kro_ref, kf_ref, o_ref):
    neigh = (p0_ref[...] + p1_ref[...]) * rx_ref[...]
    z = (jnp.dot(h_ref[...], bs_ref[...], preferred_element_type=jnp.float32)
         + jnp.dot(neigh, bn_ref[...], preferred_element_type=jnp.float32)
         + bp_ref[...])
    o_ref[...] = (jnp.dot(z, kro_ref[...], preferred_element_type=jnp.float32)
                  + jnp.dot(h0_ref[...], kf_ref[...],
                            preferred_element_type=jnp.float32))


def _readout(accf, rxf, hf, h0f, bs, bn, bp, kro, kf):
    return pl.pallas_call(
        _readout_body,
        grid=(NGB,),
        in_specs=[
            _B128(), _B128H(), _B128(), _B128(), _B128(),
            _BW(), _BW(),
            pl.BlockSpec((1, 128), lambda i: (0, 0)),
            _BW(), _BW(),
        ],
        out_specs=pl.BlockSpec((BH, 128), lambda i: (i, 0)),
        out_shape=jax.ShapeDtypeStruct((NF, 128), jnp.float32),
    )(accf, accf, rxf, hf, h0f, bs, bn, bp, kro, kf)


def _pack16(w):
    return jnp.kron(jnp.eye(8, dtype=jnp.float32), w)


def kernel(features, edge_index, e_feat,
           W_self0, W_neigh0, b0,
           W_self1, W_neigh1, b1,
           W_self2, W_neigh2, b2,
           W_ro, b_ro):
    src = edge_index[0]
    dst = edge_index[1]
    pad = EPAD - E
    ar = jnp.arange(pad, dtype=jnp.int32)
    src_p = jnp.concatenate([src, ar % N]).reshape(EPAD // C, C)
    dst_p = jnp.concatenate([dst, N + ar % (NACC - N)]).reshape(EPAD // C, C)
    w_p = jnp.concatenate([e_feat[:, 0], jnp.zeros((pad,), jnp.float32)]
                          ).reshape(EPAD // C, C)

    z2d = jnp.zeros((NACC, DH), jnp.float32)
    z1d = jnp.zeros((NACC,), jnp.float32)
    h0 = jnp.concatenate([
        jnp.pad(features, ((0, NACC - N), (0, 0))),
        jnp.ones((NACC, 7), jnp.float32),
        jnp.zeros((NACC, DH - D_IN - 7), jnp.float32)], axis=1)

    Ws0 = jnp.pad(W_self0, ((0, DH - D_IN - 7), (0, 0)))
    Wn0 = jnp.pad(W_neigh0, ((0, DH - D_IN - 7), (0, 0)))
    bpack = lambda b: jnp.tile(b, 8).reshape(1, 128)
    ones16r = jnp.ones((1, DH), jnp.float32)
    wfx = jnp.concatenate(
        [W_ro[:D_IN], b_ro.reshape(1, 1),
         jnp.zeros((DH - D_IN - 1, 1), jnp.float32)], axis=0)
    kro = _pack16(W_ro[D_IN:] @ ones16r)
    kf = _pack16(wfx @ ones16r)

    flat = lambda a: a.reshape(2 * NF, 128)
    acc0, degx, h0c = _edge_pass_deg(h0, src_p, dst_p, w_p, z2d, z1d)
    h0f = h0c.reshape(NF, 128)
    rxf = _rx(flat(degx))
    h1f = _combine(flat(acc0), rxf, h0f, _pack16(Ws0), _pack16(Wn0),
                   bpack(b0), sig=True)
    acc1 = _edge_pass(h1f.reshape(NACC, DH), src_p, dst_p, w_p, z2d)
    h2f = _combine(flat(acc1), rxf, h1f, _pack16(W_self1),
                   _pack16(W_neigh1), bpack(b1), sig=True)
    acc2 = _edge_pass(h2f.reshape(NACC, DH), src_p, dst_p, w_p, z2d)
    outf = _readout(flat(acc2), rxf, h2f, h0f, _pack16(W_self2),
                    _pack16(W_neigh2), bpack(b2), kro, kf)
    return outf.reshape(NACC, DH)[:N, :1]


# speedup vs baseline: 33.5892x; 1.2592x over previous
import functools

import jax
import jax.numpy as jnp
from jax import lax
from jax.experimental import pallas as pl
from jax.experimental.pallas import tpu as pltpu
from jax.experimental.pallas import tpu_sc as plsc

N = 100000
E = 1600000
D_IN = 3
DH = 16

NACC = 102400
C = 128
NW = 32
NCH = 400
EPAD = NW * C * NCH
EPT = EPAD // NW
RPT = NACC // 16
NR = 8
NF = NACC * DH // 128
NB = NF // 128


def _mk_edge_body(with_deg):
  def body(refs):
    if with_deg:
      (h_hbm, src_hbm, dst_hbm, w_hbm, z2_hbm, z1_hbm,
       acc_out, degx_out,
       srcv, dstv, wv, rows, ones_v, degc_v, degx_v, acc_sh, deg_sh,
       sgs, srs, sis, dg0, dg1) = refs
      dg = [dg0, dg1]
    else:
      (h_hbm, src_hbm, dst_hbm, w_hbm, z2_hbm,
       acc_out,
       srcv, dstv, wv, rows, acc_sh,
       sgs, srs, sis) = refs

    c = lax.axis_index("c")
    s = lax.axis_index("s")
    wid = c * 16 + s
    pltpu.sync_copy(z2_hbm.at[pl.ds(s * RPT, RPT)],
                    acc_sh.at[pl.ds(s * RPT, RPT)])
    if with_deg:
      pltpu.sync_copy(z1_hbm.at[pl.ds(s * RPT, RPT)],
                      deg_sh.at[pl.ds(s * RPT, RPT)])

      def _ob(j, carry):
        ones_v[pl.ds(j * 16, 16)] = jnp.ones((16,), jnp.float32)
        return carry
      lax.fori_loop(0, C // 16, _ob, 0)
    plsc.subcore_barrier()

    row0 = wid * NCH

    def idx_issue(cc, m):
      pltpu.async_copy(src_hbm.at[row0 + cc], srcv.at[m], sis.at[m])
      pltpu.async_copy(dst_hbm.at[row0 + cc], dstv.at[m], sis.at[m])
      pltpu.async_copy(w_hbm.at[row0 + cc], wv.at[m], sis.at[m])

    def idx_wait(cc, m):
      pltpu.make_async_copy(src_hbm.at[row0 + cc], srcv.at[m],
                            sis.at[m]).wait()
      pltpu.make_async_copy(dst_hbm.at[row0 + cc], dstv.at[m],
                            sis.at[m]).wait()
      pltpu.make_async_copy(w_hbm.at[row0 + cc], wv.at[m],
                            sis.at[m]).wait()

    def gather_issue(cc, m):
      pltpu.async_copy(h_hbm.at[srcv.at[m]], rows.at[m], sgs.at[m])

    def gather_wait(cc, m):
      pltpu.make_async_copy(h_hbm.at[srcv.at[m]], rows.at[m],
                            sgs.at[m]).wait()

    def scat_issue(cc, m):
      pltpu.async_copy(rows.at[m], acc_sh.at[dstv.at[m]], srs.at[m],
                       add=True)

    def scat_wait(cc, m):
      pltpu.make_async_copy(rows.at[m], acc_sh.at[dstv.at[m]],
                            srs.at[m]).wait()

    for m in range(6):
      idx_issue(m, m)
    for m in range(4):
      idx_wait(m, m)
      gather_issue(m, m)

    def _octet(i, carry):
      for j in range(NR):
        cc = i * NR + j
        m = j
        gather_wait(cc, m)

        def _mul(g, carry2):
          wvec = wv[m, pl.ds(g * 16, 16)]
          for e in range(16):
            r = g * 16 + e
            rows[m, r, :] = rows[m, r, :] * wvec[e]
          return carry2
        lax.fori_loop(0, C // 16, _mul, 0)
        scat_issue(cc, m)
        if with_deg:
          q = j & 1
          deg_wait = pltpu.make_async_copy(
              ones_v, deg_sh.at[dstv.at[m]], dg[q]).wait
          if j < 2:
            pl.when(i >= 1)(deg_wait)
          else:
            deg_wait()
          pltpu.async_copy(ones_v, deg_sh.at[dstv.at[m]], dg[q], add=True)
        sw = functools.partial(scat_wait, cc - 2, (m + 6) % NR)
        if j < 2:
          pl.when(i >= 1)(sw)
        else:
          sw()
        ii = functools.partial(idx_issue, cc + 6, (m + 6) % NR)
        if j < 2:
          ii()
        else:
          pl.when(i <= NCH // NR - 2)(ii)
        def gi():
          idx_wait(cc + 4, (m + 4) % NR)
          gather_issue(cc + 4, (m + 4) % NR)
        if j < 4:
          gi()
        else:
          pl.when(i <= NCH // NR - 2)(gi)
      return carry
    lax.fori_loop(0, NCH // NR, _octet, 0)
    scat_wait(NCH - 2, (NCH - 2) % NR)
    scat_wait(NCH - 1, (NCH - 1) % NR)
    if with_deg:
      pltpu.make_async_copy(ones_v, deg_sh.at[dstv.at[(NCH - 2) % NR]],
                            dg[0]).wait()
      pltpu.make_async_copy(ones_v, deg_sh.at[dstv.at[(NCH - 1) % NR]],
                            dg[1]).wait()
    plsc.subcore_barrier()
    pltpu.sync_copy(acc_sh.at[pl.ds(s * RPT, RPT)],
                    acc_out.at[c, pl.ds(s * RPT, RPT)])
    if with_deg:
      ones16 = jnp.ones((16,), jnp.float32)

      def _exp(t, carry):
        base = s * RPT + t * 128
        pltpu.sync_copy(deg_sh.at[pl.ds(base, 128)], degc_v)

        def _g(g, cc):
          dv = degc_v[pl.ds(g * 16, 16)]
          for e in range(16):
            degx_v[g * 16 + e, :] = ones16 * dv[e]
          return cc
        lax.fori_loop(0, 8, _g, 0)
        pltpu.sync_copy(degx_v, degx_out.at[c, pl.ds(base, 128)])
        return carry
      lax.fori_loop(0, RPT // 128, _exp, 0)
  return body


def _edge_pass_deg_body(*refs):
  return _mk_edge_body(True)(refs)


def _edge_pass_body(*refs):
  return _mk_edge_body(False)(refs)


_IDX_SCRATCH = lambda dt: pltpu.VMEM((RB, C), dt)
_ROWS_SCRATCH = lambda: pltpu.VMEM((C, DH), jnp.float32)


@functools.cache
def _build_edge_pass_deg():
  return functools.partial(
      pl.kernel,
      mesh=plsc.VectorSubcoreMesh(core_axis_name="c", subcore_axis_name="s"),
      compiler_params=pltpu.CompilerParams(use_tc_tiling_on_sc=False),
      out_type=(
          jax.ShapeDtypeStruct((2, NACC, DH), jnp.float32),
          jax.ShapeDtypeStruct((2, NACC, DH), jnp.float32),
      ),
      scratch_types=[
          pltpu.VMEM((NR, C), jnp.int32),
          pltpu.VMEM((NR, C), jnp.int32),
          pltpu.VMEM((NR, C), jnp.float32),
          pltpu.VMEM((NR, C, DH), jnp.float32),
          pltpu.VMEM((C,), jnp.float32),
          pltpu.VMEM((128,), jnp.float32),
          pltpu.VMEM((128, DH), jnp.float32),
          pltpu.VMEM_SHARED((NACC, DH), jnp.float32),
          pltpu.VMEM_SHARED((NACC,), jnp.float32),
          pltpu.SemaphoreType.DMA((NR,)),
          pltpu.SemaphoreType.DMA((NR,)),
          pltpu.SemaphoreType.DMA((NR,)),
          pltpu.SemaphoreType.DMA,
          pltpu.SemaphoreType.DMA,
      ],
  )(_edge_pass_deg_body)


def _edge_pass_deg(*args):
  return _build_edge_pass_deg()(*args)


@functools.cache
def _build_edge_pass():
  return functools.partial(
      pl.kernel,
      mesh=plsc.VectorSubcoreMesh(core_axis_name="c", subcore_axis_name="s"),
      compiler_params=pltpu.CompilerParams(use_tc_tiling_on_sc=False),
      out_type=jax.ShapeDtypeStruct((2, NACC, DH), jnp.float32),
      scratch_types=[
          pltpu.VMEM((NR, C), jnp.int32),
          pltpu.VMEM((NR, C), jnp.int32),
          pltpu.VMEM((NR, C), jnp.float32),
          pltpu.VMEM((NR, C, DH), jnp.float32),
          pltpu.VMEM_SHARED((NACC, DH), jnp.float32),
          pltpu.SemaphoreType.DMA((NR,)),
          pltpu.SemaphoreType.DMA((NR,)),
          pltpu.SemaphoreType.DMA((NR,)),
      ],
  )(_edge_pass_body)


def _edge_pass(*args):
  return _build_edge_pass()(*args)


def _rx_body(d0_ref, d1_ref, o_ref):
    o_ref[...] = 1.0 / jnp.maximum(d0_ref[...] + d1_ref[...], 1.0)


def _rx(degxf):
    return pl.pallas_call(
        _rx_body,
        grid=(NGB,),
        in_specs=[_B128(), _B128H()],
        out_specs=pl.BlockSpec((BH, 128), lambda i: (i, 0)),
        out_shape=jax.ShapeDtypeStruct((NF, 128), jnp.float32),
    )(degxf, degxf)


def _combine_body(sig, p0_ref, p1_ref, rx_ref, h_ref, bs_ref,
                  bn_ref, bp_ref, o_ref):
    neigh = (p0_ref[...] + p1_ref[...]) * rx_ref[...]
    z = (jnp.dot(h_ref[...], bs_ref[...], preferred_element_type=jnp.float32)
         + jnp.dot(neigh, bn_ref[...], preferred_element_type=jnp.float32)
         + bp_ref[...])
    if sig:
        z = 1.0 / (1.0 + jnp.exp(-z))
    o_ref[...] = z


BH = 1280
NGB = NF // BH
_B128 = lambda: pl.BlockSpec((BH, 128), lambda i: (i, 0))
_B128H = lambda: pl.BlockSpec((BH, 128), lambda i: (i + NGB, 0))
_BW = lambda: pl.BlockSpec((128, 128), lambda i: (0, 0))


def _combine(accf, rxf, hf, bs, bn, bp, sig):
    return pl.pallas_call(
        functools.partial(_combine_body, sig),
        grid=(NGB,),
        in_specs=[
            _B128(), _B128H(), _B128(), _B128(),
            _BW(), _BW(),
            pl.BlockSpec((1, 128), lambda i: (0, 0)),
        ],
        out_specs=pl.BlockSpec((BH, 128), lambda i: (i, 0)),
        out_shape=jax.ShapeDtypeStruct((NF, 128), jnp.float32),
    )(accf, accf, rxf, hf, bs, bn, bp)


def _readout_body(p0_ref, p1_ref, rx_ref, h_ref, h0_ref, bs_ref,
                  bn_ref, bp_ref, kro_ref, kf_ref, o_ref):
    neigh = (p0_ref[...] + p1_ref[...]) * rx_ref[...]
    z = (jnp.dot(h_ref[...], bs_ref[...], preferred_element_type=jnp.float32)
         + jnp.dot(neigh, bn_ref[...], preferred_element_type=jnp.float32)
         + bp_ref[...])
    o_ref[...] = (jnp.dot(z, kro_ref[...], preferred_element_type=jnp.float32)
                  + jnp.dot(h0_ref[...], kf_ref[...],
                            preferred_element_type=jnp.float32))


def _readout(accf, rxf, hf, h0f, bs, bn, bp, kro, kf):
    return pl.pallas_call(
        _readout_body,
        grid=(NGB,),
        in_specs=[
            _B128(), _B128H(), _B128(), _B128(), _B128(),
            _BW(), _BW(),
            pl.BlockSpec((1, 128), lambda i: (0, 0)),
            _BW(), _BW(),
        ],
        out_specs=pl.BlockSpec((BH, 128), lambda i: (i, 0)),
        out_shape=jax.ShapeDtypeStruct((NF, 128), jnp.float32),
    )(accf, accf, rxf, hf, h0f, bs, bn, bp, kro, kf)


def _pack16(w):
    return jnp.kron(jnp.eye(8, dtype=jnp.float32), w)


def kernel(features, edge_index, e_feat,
           W_self0, W_neigh0, b0,
           W_self1, W_neigh1, b1,
           W_self2, W_neigh2, b2,
           W_ro, b_ro):
    src = edge_index[0]
    dst = edge_index[1]
    pad = EPAD - E
    ar = jnp.arange(pad, dtype=jnp.int32)
    src_p = jnp.concatenate([src, ar % N]).reshape(EPAD // C, C)
    dst_p = jnp.concatenate([dst, N + ar % (NACC - N)]).reshape(EPAD // C, C)
    w_p = jnp.concatenate([e_feat[:, 0], jnp.zeros((pad,), jnp.float32)]
                          ).reshape(EPAD // C, C)

    z2d = jnp.zeros((NACC, DH), jnp.float32)
    z1d = jnp.zeros((NACC,), jnp.float32)
    h0 = jnp.concatenate([
        jnp.pad(features, ((0, NACC - N), (0, 0))),
        jnp.ones((NACC, 7), jnp.float32),
        jnp.zeros((NACC, DH - D_IN - 7), jnp.float32)], axis=1)

    Ws0 = jnp.pad(W_self0, ((0, DH - D_IN - 7), (0, 0)))
    Wn0 = jnp.pad(W_neigh0, ((0, DH - D_IN - 7), (0, 0)))
    bpack = lambda b: jnp.tile(b, 8).reshape(1, 128)
    ones16r = jnp.ones((1, DH), jnp.float32)
    wfx = jnp.concatenate(
        [W_ro[:D_IN], b_ro.reshape(1, 1),
         jnp.zeros((DH - D_IN - 1, 1), jnp.float32)], axis=0)
    kro = _pack16(W_ro[D_IN:] @ ones16r)
    kf = _pack16(wfx @ ones16r)

    flat = lambda a: a.reshape(2 * NF, 128)
    h0f = h0.reshape(NF, 128)
    acc0, degx = _edge_pass_deg(h0, src_p, dst_p, w_p, z2d, z1d)
    rxf = _rx(flat(degx))
    h1f = _combine(flat(acc0), rxf, h0f, _pack16(Ws0), _pack16(Wn0),
                   bpack(b0), sig=True)
    acc1 = _edge_pass(h1f.reshape(NACC, DH), src_p, dst_p, w_p, z2d)
    h2f = _combine(flat(acc1), rxf, h1f, _pack16(W_self1),
                   _pack16(W_neigh1), bpack(b1), sig=True)
    acc2 = _edge_pass(h2f.reshape(NACC, DH), src_p, dst_p, w_p, z2d)
    outf = _readout(flat(acc2), rxf, h2f, h0f, _pack16(W_self2),
                    _pack16(W_neigh2), bpack(b2), kro, kf)
    return outf.reshape(NACC, DH)[:N, :1]


# speedup vs baseline: 35.8666x; 1.0678x over previous
import functools

import jax
import jax.numpy as jnp
from jax import lax
from jax.experimental import pallas as pl
from jax.experimental.pallas import tpu as pltpu
from jax.experimental.pallas import tpu_sc as plsc

N = 100000
E = 1600000
D_IN = 3
DH = 16

NACC = 102400
C = 128
NW = 32
NCHM = 390
EMAIN = NW * C * NCHM
NXTRA = (E - EMAIN) // C
EPT = C * NCHM
RPT = NACC // 16
NR = 8
NF = NACC * DH // 128
NB = NF // 128


def _mk_edge_body(with_deg):
  def body(refs):
    if with_deg:
      (h_hbm, ei_hbm, w_hbm, z2_hbm, z1_hbm,
       acc_out, degx_out,
       srcv, dstv, wv, rows, ones_v, degc_v, degx_v, acc_sh, deg_sh,
       sgs, srs, sis, dg0, dg1) = refs
      dg = [dg0, dg1]
    else:
      (h_hbm, ei_hbm, w_hbm, z2_hbm,
       acc_out,
       srcv, dstv, wv, rows, acc_sh,
       sgs, srs, sis) = refs

    c = lax.axis_index("c")
    s = lax.axis_index("s")
    wid = c * 16 + s
    pltpu.sync_copy(z2_hbm.at[pl.ds(s * RPT, RPT)],
                    acc_sh.at[pl.ds(s * RPT, RPT)])
    if with_deg:
      pltpu.sync_copy(z1_hbm.at[pl.ds(s * RPT, RPT)],
                      deg_sh.at[pl.ds(s * RPT, RPT)])

      def _ob(j, carry):
        ones_v[pl.ds(j * 16, 16)] = jnp.ones((16,), jnp.float32)
        return carry
      lax.fori_loop(0, C // 16, _ob, 0)
    plsc.subcore_barrier()

    tbase = wid * EPT

    def idx_issue(cc, m, base=None):
      b = tbase + cc * C if base is None else base
      pltpu.async_copy(ei_hbm.at[0, pl.ds(b, C)], srcv.at[m], sis.at[m])
      pltpu.async_copy(ei_hbm.at[1, pl.ds(b, C)], dstv.at[m], sis.at[m])
      pltpu.async_copy(w_hbm.at[pl.ds(b, C)], wv.at[m], sis.at[m])

    def idx_wait(cc, m, base=None):
      b = tbase + cc * C if base is None else base
      pltpu.make_async_copy(ei_hbm.at[0, pl.ds(b, C)], srcv.at[m],
                            sis.at[m]).wait()
      pltpu.make_async_copy(ei_hbm.at[1, pl.ds(b, C)], dstv.at[m],
                            sis.at[m]).wait()
      pltpu.make_async_copy(w_hbm.at[pl.ds(b, C)], wv.at[m],
                            sis.at[m]).wait()

    def gather_issue(cc, m):
      pltpu.async_copy(h_hbm.at[srcv.at[m]], rows.at[m], sgs.at[m])

    def gather_wait(cc, m):
      pltpu.make_async_copy(h_hbm.at[srcv.at[m]], rows.at[m],
                            sgs.at[m]).wait()

    def scat_issue(cc, m):
      pltpu.async_copy(rows.at[m], acc_sh.at[dstv.at[m]], srs.at[m],
                       add=True)

    def scat_wait(cc, m):
      pltpu.make_async_copy(rows.at[m], acc_sh.at[dstv.at[m]],
                            srs.at[m]).wait()

    def mult(m):
      def _mul(g, carry2):
        wvec = wv[m, pl.ds(g * 16, 16)]
        for e in range(16):
          r = g * 16 + e
          rows[m, r, :] = rows[m, r, :] * wvec[e]
        return carry2
      lax.fori_loop(0, C // 16, _mul, 0)

    def deg_issue(m, q):
      pltpu.async_copy(ones_v, deg_sh.at[dstv.at[m]], dg[q], add=True)

    def deg_wait(m, q):
      pltpu.make_async_copy(ones_v, deg_sh.at[dstv.at[m]], dg[q]).wait()

    for m in range(6):
      idx_issue(m, m)
    for m in range(4):
      idx_wait(m, m)
      gather_issue(m, m)

    def _octet(i, carry):
      for j in range(NR):
        cc = i * NR + j
        m = j
        gather_wait(cc, m)
        mult(m)
        scat_issue(cc, m)
        if with_deg:
          q = j & 1
          if j < 2:
            pl.when(i >= 1)(lambda: deg_wait(m, q))
          else:
            deg_wait(m, q)
          deg_issue(m, q)
        sw = functools.partial(scat_wait, cc - 2, (m + 6) % NR)
        if j < 2:
          pl.when(i >= 1)(sw)
        else:
          sw()
        idx_issue(cc + 6, (m + 6) % NR)
        idx_wait(cc + 4, (m + 4) % NR)
        gather_issue(cc + 4, (m + 4) % NR)
      return carry
    lax.fori_loop(0, NCHM // NR, _octet, 0)

    for cc in range(NCHM // NR * NR, NCHM):
      m = cc % NR
      gather_wait(cc, m)
      mult(m)
      scat_issue(cc, m)
      if with_deg:
        deg_wait(m, cc & 1)
        deg_issue(m, cc & 1)
      scat_wait(cc - 2, (m + 6) % NR)
      if cc + 4 < NCHM:
        idx_wait(cc + 4, (m + 4) % NR)
        gather_issue(cc + 4, (m + 4) % NR)
    scat_wait(NCHM - 2, (NCHM - 2) % NR)
    scat_wait(NCHM - 1, (NCHM - 1) % NR)
    if with_deg:
      deg_wait((NCHM - 2) % NR, 0)
      deg_wait((NCHM - 1) % NR, 1)

    @pl.when(wid < NXTRA)
    def _extra():
      base = EMAIN + wid * C
      idx_issue(0, 0, base=base)
      idx_wait(0, 0, base=base)
      pltpu.async_copy(h_hbm.at[srcv.at[0]], rows.at[0], sgs.at[0])
      pltpu.make_async_copy(h_hbm.at[srcv.at[0]], rows.at[0],
                            sgs.at[0]).wait()
      mult(0)
      pltpu.async_copy(rows.at[0], acc_sh.at[dstv.at[0]], srs.at[0],
                       add=True)
      pltpu.make_async_copy(rows.at[0], acc_sh.at[dstv.at[0]],
                            srs.at[0]).wait()
      if with_deg:
        deg_issue(0, 0)
        deg_wait(0, 0)

    plsc.subcore_barrier()
    pltpu.sync_copy(acc_sh.at[pl.ds(s * RPT, RPT)],
                    acc_out.at[c, pl.ds(s * RPT, RPT)])
    if with_deg:
      ones16 = jnp.ones((16,), jnp.float32)

      def _exp(t, carry):
        base = s * RPT + t * 128
        pltpu.sync_copy(deg_sh.at[pl.ds(base, 128)], degc_v)

        def _g(g, cc):
          dv = degc_v[pl.ds(g * 16, 16)]
          for e in range(16):
            degx_v[g * 16 + e, :] = ones16 * dv[e]
          return cc
        lax.fori_loop(0, 8, _g, 0)
        pltpu.sync_copy(degx_v, degx_out.at[c, pl.ds(base, 128)])
        return carry
      lax.fori_loop(0, RPT // 128, _exp, 0)
  return body


def _edge_pass_deg_body(*refs):
  return _mk_edge_body(True)(refs)


def _edge_pass_body(*refs):
  return _mk_edge_body(False)(refs)


_IDX_SCRATCH = lambda dt: pltpu.VMEM((RB, C), dt)
_ROWS_SCRATCH = lambda: pltpu.VMEM((C, DH), jnp.float32)


@functools.cache
def _build_edge_pass_deg():
  return functools.partial(
      pl.kernel,
      mesh=plsc.VectorSubcoreMesh(core_axis_name="c", subcore_axis_name="s"),
      compiler_params=pltpu.CompilerParams(use_tc_tiling_on_sc=False),
      out_type=(
          jax.ShapeDtypeStruct((2, NACC, DH), jnp.float32),
          jax.ShapeDtypeStruct((2, NACC, DH), jnp.float32),
      ),
      scratch_types=[
          pltpu.VMEM((NR, C), jnp.int32),
          pltpu.VMEM((NR, C), jnp.int32),
          pltpu.VMEM((NR, C), jnp.float32),
          pltpu.VMEM((NR, C, DH), jnp.float32),
          pltpu.VMEM((C,), jnp.float32),
          pltpu.VMEM((128,), jnp.float32),
          pltpu.VMEM((128, DH), jnp.float32),
          pltpu.VMEM_SHARED((NACC, DH), jnp.float32),
          pltpu.VMEM_SHARED((NACC,), jnp.float32),
          pltpu.SemaphoreType.DMA((NR,)),
          pltpu.SemaphoreType.DMA((NR,)),
          pltpu.SemaphoreType.DMA((NR,)),
          pltpu.SemaphoreType.DMA,
          pltpu.SemaphoreType.DMA,
      ],
  )(_edge_pass_deg_body)


def _edge_pass_deg(*args):
  return _build_edge_pass_deg()(*args)


@functools.cache
def _build_edge_pass():
  return functools.partial(
      pl.kernel,
      mesh=plsc.VectorSubcoreMesh(core_axis_name="c", subcore_axis_name="s"),
      compiler_params=pltpu.CompilerParams(use_tc_tiling_on_sc=False),
      out_type=jax.ShapeDtypeStruct((2, NACC, DH), jnp.float32),
      scratch_types=[
          pltpu.VMEM((NR, C), jnp.int32),
          pltpu.VMEM((NR, C), jnp.int32),
          pltpu.VMEM((NR, C), jnp.float32),
          pltpu.VMEM((NR, C, DH), jnp.float32),
          pltpu.VMEM_SHARED((NACC, DH), jnp.float32),
          pltpu.SemaphoreType.DMA((NR,)),
          pltpu.SemaphoreType.DMA((NR,)),
          pltpu.SemaphoreType.DMA((NR,)),
      ],
  )(_edge_pass_body)


def _edge_pass(*args):
  return _build_edge_pass()(*args)


def _rx_body(d0_ref, d1_ref, o_ref):
    o_ref[...] = 1.0 / jnp.maximum(d0_ref[...] + d1_ref[...], 1.0)


def _rx(degxf):
    return pl.pallas_call(
        _rx_body,
        grid=(NGB,),
        in_specs=[_B128(), _B128H()],
        out_specs=pl.BlockSpec((BH, 128), lambda i: (i, 0)),
        out_shape=jax.ShapeDtypeStruct((NF, 128), jnp.float32),
    )(degxf, degxf)


def _combine_body(sig, p0_ref, p1_ref, rx_ref, h_ref, bs_ref,
                  bn_ref, bp_ref, o_ref):
    neigh = (p0_ref[...] + p1_ref[...]) * rx_ref[...]
    z = (jnp.dot(h_ref[...], bs_ref[...], preferred_element_type=jnp.float32)
         + jnp.dot(neigh, bn_ref[...], preferred_element_type=jnp.float32)
         + bp_ref[...])
    if sig:
        z = 1.0 / (1.0 + jnp.exp(-z))
    o_ref[...] = z


BH = 1280
NGB = NF // BH
_B128 = lambda: pl.BlockSpec((BH, 128), lambda i: (i, 0))
_B128H = lambda: pl.BlockSpec((BH, 128), lambda i: (i + NGB, 0))
_BW = lambda: pl.BlockSpec((128, 128), lambda i: (0, 0))


def _combine(accf, rxf, hf, bs, bn, bp, sig):
    return pl.pallas_call(
        functools.partial(_combine_body, sig),
        grid=(NGB,),
        in_specs=[
            _B128(), _B128H(), _B128(), _B128(),
            _BW(), _BW(),
            pl.BlockSpec((1, 128), lambda i: (0, 0)),
        ],
        out_specs=pl.BlockSpec((BH, 128), lambda i: (i, 0)),
        out_shape=jax.ShapeDtypeStruct((NF, 128), jnp.float32),
    )(accf, accf, rxf, hf, bs, bn, bp)


def _readout_body(p0_ref, p1_ref, rx_ref, h_ref, h0_ref, bs_ref,
                  bn_ref, bp_ref, kro_ref, kf_ref, o_ref):
    neigh = (p0_ref[...] + p1_ref[...]) * rx_ref[...]
    z = (jnp.dot(h_ref[...], bs_ref[...], preferred_element_type=jnp.float32)
         + jnp.dot(neigh, bn_ref[...], preferred_element_type=jnp.float32)
         + bp_ref[...])
    o_ref[...] = (jnp.dot(z, kro_ref[...], preferred_element_type=jnp.float32)
                  + jnp.dot(h0_ref[...], kf_ref[...],
                            preferred_element_type=jnp.float32))


def _readout(accf, rxf, hf, h0f, bs, bn, bp, kro, kf):
    return pl.pallas_call(
        _readout_body,
        grid=(NGB,),
        in_specs=[
            _B128(), _B128H(), _B128(), _B128(), _B128(),
            _BW(), _BW(),
            pl.BlockSpec((1, 128), lambda i: (0, 0)),
            _BW(), _BW(),
        ],
        out_specs=pl.BlockSpec((BH, 128), lambda i: (i, 0)),
        out_shape=jax.ShapeDtypeStruct((NF, 128), jnp.float32),
    )(accf, accf, rxf, hf, h0f, bs, bn, bp, kro, kf)


def _pack16(w):
    return jnp.kron(jnp.eye(8, dtype=jnp.float32), w)


def kernel(features, edge_index, e_feat,
           W_self0, W_neigh0, b0,
           W_self1, W_neigh1, b1,
           W_self2, W_neigh2, b2,
           W_ro, b_ro):
    w1 = e_feat[:, 0]
    z2d = jnp.zeros((NACC, DH), jnp.float32)
    z1d = jnp.zeros((NACC,), jnp.float32)
    h0 = jnp.concatenate([
        jnp.pad(features, ((0, NACC - N), (0, 0))),
        jnp.ones((NACC, 7), jnp.float32),
        jnp.zeros((NACC, DH - D_IN - 7), jnp.float32)], axis=1)

    Ws0 = jnp.pad(W_self0, ((0, DH - D_IN - 7), (0, 0)))
    Wn0 = jnp.pad(W_neigh0, ((0, DH - D_IN - 7), (0, 0)))
    bpack = lambda b: jnp.tile(b, 8).reshape(1, 128)
    ones16r = jnp.ones((1, DH), jnp.float32)
    wfx = jnp.concatenate(
        [W_ro[:D_IN], b_ro.reshape(1, 1),
         jnp.zeros((DH - D_IN - 1, 1), jnp.float32)], axis=0)
    kro = _pack16(W_ro[D_IN:] @ ones16r)
    kf = _pack16(wfx @ ones16r)

    flat = lambda a: a.reshape(2 * NF, 128)
    h0f = h0.reshape(NF, 128)
    acc0, degx = _edge_pass_deg(h0, edge_index, w1, z2d, z1d)
    rxf = _rx(flat(degx))
    h1f = _combine(flat(acc0), rxf, h0f, _pack16(Ws0), _pack16(Wn0),
                   bpack(b0), sig=True)
    acc1 = _edge_pass(h1f.reshape(NACC, DH), edge_index, w1, z2d)
    h2f = _combine(flat(acc1), rxf, h1f, _pack16(W_self1),
                   _pack16(W_neigh1), bpack(b1), sig=True)
    acc2 = _edge_pass(h2f.reshape(NACC, DH), edge_index, w1, z2d)
    outf = _readout(flat(acc2), rxf, h2f, h0f, _pack16(W_self2),
                    _pack16(W_neigh2), bpack(b2), kro, kf)
    return outf.reshape(NACC, DH)[:N, :1]
